# Initial kernel scaffold; baseline (speedup 1.0000x reference)
import jax
import jax.numpy as jnp
from jax.experimental import pallas as pl


def kernel(x, adj_t, pos, batch, sub_x, sub_adj_t, sub_batch, edge_index, batch_lengths, edge_batch, We1, be1, We2, be2, gamma, beta, Wg1, bg1, Wg2, bg2):
    raise NotImplementedError("write your pallas kernel here")


# speedup vs baseline: 1.0235x; 1.0235x over previous
import functools

import jax
import jax.numpy as jnp
from jax.experimental import pallas as pl
from jax.experimental.pallas import tpu as pltpu

N, D, LAT, NS, ES, E, G = 10000, 128, 128, 100000, 400000, 160000, 64
F = D + LAT


def _mm_bias_body(a_ref, w_ref, b_ref, o_ref, *, relu):
    acc = jnp.dot(a_ref[...], w_ref[...], preferred_element_type=jnp.float32)
    acc = acc + b_ref[...]
    if relu:
        acc = jnp.maximum(acc, 0.0)
    o_ref[...] = acc


def mm_bias(a, w, b, relu, block_m=2000):
    m, k = a.shape
    n = w.shape[1]
    assert m % block_m == 0
    return pl.pallas_call(
        functools.partial(_mm_bias_body, relu=relu),
        grid=(m // block_m,),
        in_specs=[
            pl.BlockSpec((block_m, k), lambda i: (i, 0)),
            pl.BlockSpec((k, n), lambda i: (0, 0)),
            pl.BlockSpec((1, n), lambda i: (0, 0)),
        ],
        out_specs=pl.BlockSpec((block_m, n), lambda i: (i, 0)),
        out_shape=jax.ShapeDtypeStruct((m, n), jnp.float32),
    )(a, w, b.reshape(1, n))


def _bn_stats_body(x_ref, zs_ref, cnt_ref, xz_ref, sums_ref, sq_ref,
                   acc_s, acc_q, *, nsteps):
    i = pl.program_id(0)

    @pl.when(i == 0)
    def _():
        acc_s[...] = jnp.zeros_like(acc_s)
        acc_q[...] = jnp.zeros_like(acc_q)

    z = zs_ref[...] / jnp.maximum(cnt_ref[...], 1.0)
    xz = jnp.concatenate((x_ref[...], z), axis=1)
    xz_ref[...] = xz
    acc_s[...] += jnp.sum(xz, axis=0, keepdims=True)
    acc_q[...] += jnp.sum(xz * xz, axis=0, keepdims=True)

    @pl.when(i == nsteps - 1)
    def _():
        sums_ref[...] = acc_s[...]
        sq_ref[...] = acc_q[...]


def bn_stats(x, zs, cnt, block_m=2000):
    nsteps = N // block_m
    return pl.pallas_call(
        functools.partial(_bn_stats_body, nsteps=nsteps),
        grid=(nsteps,),
        in_specs=[
            pl.BlockSpec((block_m, D), lambda i: (i, 0)),
            pl.BlockSpec((block_m, LAT), lambda i: (i, 0)),
            pl.BlockSpec((block_m, 1), lambda i: (i, 0)),
        ],
        out_specs=[
            pl.BlockSpec((block_m, F), lambda i: (i, 0)),
            pl.BlockSpec((1, F), lambda i: (0, 0)),
            pl.BlockSpec((1, F), lambda i: (0, 0)),
        ],
        out_shape=[
            jax.ShapeDtypeStruct((N, F), jnp.float32),
            jax.ShapeDtypeStruct((1, F), jnp.float32),
            jax.ShapeDtypeStruct((1, F), jnp.float32),
        ],
        scratch_shapes=[
            pltpu.VMEM((1, F), jnp.float32),
            pltpu.VMEM((1, F), jnp.float32),
        ],
    )(x, zs, cnt.reshape(N, 1))


def _bn_apply_body(xz_ref, s_ref, q_ref, g_ref, b_ref, o_ref):
    mu = s_ref[...] / N
    var = q_ref[...] / N - mu * mu
    rstd = jax.lax.rsqrt(var + 1e-5)
    o_ref[...] = (xz_ref[...] - mu) * rstd * g_ref[...] + b_ref[...]


def bn_apply(xz, sums, sq, gamma, beta, block_m=2000):
    return pl.pallas_call(
        _bn_apply_body,
        grid=(N // block_m,),
        in_specs=[
            pl.BlockSpec((block_m, F), lambda i: (i, 0)),
            pl.BlockSpec((1, F), lambda i: (0, 0)),
            pl.BlockSpec((1, F), lambda i: (0, 0)),
            pl.BlockSpec((1, F), lambda i: (0, 0)),
            pl.BlockSpec((1, F), lambda i: (0, 0)),
        ],
        out_specs=pl.BlockSpec((block_m, F), lambda i: (i, 0)),
        out_shape=jax.ShapeDtypeStruct((N, F), jnp.float32),
    )(xz, sums, sq, gamma.reshape(1, F), beta.reshape(1, F))


def _pool_out_body(hg_ref, batch_ref, w_ref, b_ref, o_ref, acc, cnt,
                   *, block_m, nsteps):
    i = pl.program_id(0)

    @pl.when(i == 0)
    def _():
        acc[...] = jnp.zeros_like(acc)
        cnt[...] = jnp.zeros_like(cnt)

    ids = batch_ref[0, 0, :]
    gids = jax.lax.broadcasted_iota(jnp.int32, (G, block_m), 0)
    onehot = (ids[None, :] == gids).astype(jnp.float32)
    acc[...] += jnp.dot(onehot, hg_ref[...],
                        preferred_element_type=jnp.float32)
    cnt[...] += jnp.sum(onehot, axis=1, keepdims=True)

    @pl.when(i == nsteps - 1)
    def _():
        pooled = acc[...] / jnp.maximum(cnt[...], 1.0)
        o_ref[...] = jnp.dot(pooled, w_ref[...],
                             preferred_element_type=jnp.float32) + b_ref[...]


def pool_out(hg, batch, w, b, block_m=2000):
    nsteps = N // block_m
    return pl.pallas_call(
        functools.partial(_pool_out_body, block_m=block_m, nsteps=nsteps),
        grid=(nsteps,),
        in_specs=[
            pl.BlockSpec((block_m, F), lambda i: (i, 0)),
            pl.BlockSpec((1, 1, block_m), lambda i: (i, 0, 0)),
            pl.BlockSpec((F, 128), lambda i: (0, 0)),
            pl.BlockSpec((1, 128), lambda i: (0, 0)),
        ],
        out_specs=pl.BlockSpec((G, 128), lambda i: (0, 0)),
        out_shape=jax.ShapeDtypeStruct((G, 128), jnp.float32),
        scratch_shapes=[
            pltpu.VMEM((G, F), jnp.float32),
            pltpu.VMEM((G, 1), jnp.float32),
        ],
    )(hg, batch.astype(jnp.int32).reshape(nsteps, 1, block_m), w,
      b.reshape(1, 128))


def kernel(x, adj_t, pos, batch, sub_x, sub_adj_t, sub_batch, edge_index,
           batch_lengths, edge_batch, We1, be1, We2, be2, gamma, beta,
           Wg1, bg1, Wg2, bg2):
    h = mm_bias(sub_x, We1, be1, relu=True)
    s, d = sub_adj_t[0], sub_adj_t[1]
    agg = jax.ops.segment_sum(h[s], d, num_segments=NS)
    h2 = mm_bias(agg, We2, be2, relu=True)
    zs = jax.ops.segment_sum(h2, sub_batch, num_segments=N)
    cnt = jax.ops.segment_sum(jnp.ones((NS,), jnp.float32), sub_batch,
                              num_segments=N)
    xz, sums, sq = bn_stats(x, zs, cnt)
    xn = bn_apply(xz, sums, sq, gamma, beta)
    es, ed = edge_index[0], edge_index[1]
    w = jnp.exp(-jnp.sum((pos[es] - pos[ed]) ** 2, axis=1))
    m = jax.ops.segment_sum(xn[es] * w[:, None], ed, num_segments=N)
    hg = mm_bias(m, Wg1, bg1, relu=True)
    return pool_out(hg, batch, Wg2, bg2)


# speedup vs baseline: 1.5870x; 1.5506x over previous
import functools

import jax
import jax.numpy as jnp
from jax import lax
from jax.experimental import pallas as pl
from jax.experimental.pallas import tpu as pltpu
from jax.experimental.pallas import tpu_sc as plsc

N, D, LAT, NS, ES, E, G = 10000, 128, 128, 100000, 400000, 160000, 64
F = D + LAT


def _mm_bias_body(a_ref, w_ref, b_ref, o_ref, *, relu):
    acc = jnp.dot(a_ref[...], w_ref[...], preferred_element_type=jnp.float32)
    acc = acc + b_ref[...]
    if relu:
        acc = jnp.maximum(acc, 0.0)
    o_ref[...] = acc


def mm_bias(a, w, b, relu, block_m=2000, out_rows=None):
    m, k = a.shape
    n = w.shape[1]
    assert m % block_m == 0
    return pl.pallas_call(
        functools.partial(_mm_bias_body, relu=relu),
        grid=(m // block_m,),
        in_specs=[
            pl.BlockSpec((block_m, k), lambda i: (i, 0)),
            pl.BlockSpec((k, n), lambda i: (0, 0)),
            pl.BlockSpec((1, n), lambda i: (0, 0)),
        ],
        out_specs=pl.BlockSpec((block_m, n), lambda i: (i, 0)),
        out_shape=jax.ShapeDtypeStruct((out_rows or m, n), jnp.float32),
    )(a, w, b.reshape(1, n))


def _bn_stats_body(x_ref, zc_ref, ct_ref, xz_ref, sums_ref, sq_ref,
                   acc_s, acc_q, *, nsteps):
    i = pl.program_id(0)

    @pl.when(i == 0)
    def _():
        acc_s[...] = jnp.zeros_like(acc_s)
        acc_q[...] = jnp.zeros_like(acc_q)

    zc = zc_ref[0] + zc_ref[1]
    ct = ct_ref[0, :, 0:1] + ct_ref[1, :, 0:1]
    z = zc / jnp.maximum(ct, 1.0)
    xz = jnp.concatenate((x_ref[...], z), axis=1)
    xz_ref[...] = xz
    acc_s[...] += jnp.sum(xz, axis=0, keepdims=True)
    acc_q[...] += jnp.sum(xz * xz, axis=0, keepdims=True)

    @pl.when(i == nsteps - 1)
    def _():
        sums_ref[...] = acc_s[...]
        sq_ref[...] = acc_q[...]


def bn_stats(x, zcnt, cnts, block_m=2000):
    nsteps = N // block_m
    return pl.pallas_call(
        functools.partial(_bn_stats_body, nsteps=nsteps),
        grid=(nsteps,),
        in_specs=[
            pl.BlockSpec((block_m, D), lambda i: (i, 0)),
            pl.BlockSpec((2, block_m, LAT), lambda i: (0, i, 0)),
            pl.BlockSpec((2, block_m, LAT), lambda i: (0, i, 0)),
        ],
        out_specs=[
            pl.BlockSpec((block_m, F), lambda i: (i, 0)),
            pl.BlockSpec((1, F), lambda i: (0, 0)),
            pl.BlockSpec((1, F), lambda i: (0, 0)),
        ],
        out_shape=[
            jax.ShapeDtypeStruct((N, F), jnp.float32),
            jax.ShapeDtypeStruct((1, F), jnp.float32),
            jax.ShapeDtypeStruct((1, F), jnp.float32),
        ],
        scratch_shapes=[
            pltpu.VMEM((1, F), jnp.float32),
            pltpu.VMEM((1, F), jnp.float32),
        ],
    )(x, zcnt, cnts)


def _bn_apply_body(xz_ref, s_ref, q_ref, g_ref, b_ref, o_ref):
    mu = s_ref[...] / N
    var = q_ref[...] / N - mu * mu
    rstd = jax.lax.rsqrt(var + 1e-5)
    o_ref[...] = (xz_ref[...] - mu) * rstd * g_ref[...] + b_ref[...]


def bn_apply(xz, sums, sq, gamma, beta, block_m=2000):
    return pl.pallas_call(
        _bn_apply_body,
        grid=(N // block_m,),
        in_specs=[
            pl.BlockSpec((block_m, F), lambda i: (i, 0)),
            pl.BlockSpec((1, F), lambda i: (0, 0)),
            pl.BlockSpec((1, F), lambda i: (0, 0)),
            pl.BlockSpec((1, F), lambda i: (0, 0)),
            pl.BlockSpec((1, F), lambda i: (0, 0)),
        ],
        out_specs=pl.BlockSpec((block_m, F), lambda i: (i, 0)),
        out_shape=jax.ShapeDtypeStruct((N, F), jnp.float32),
    )(xz, sums, sq, gamma.reshape(1, F), beta.reshape(1, F))


def _pool_out_body(hg_ref, batch_ref, w_ref, b_ref, o_ref, acc, cnt,
                   *, block_m, nsteps):
    i = pl.program_id(0)

    @pl.when(i == 0)
    def _():
        acc[...] = jnp.zeros_like(acc)
        cnt[...] = jnp.zeros_like(cnt)

    ids = batch_ref[0, 0, :]
    gids = jax.lax.broadcasted_iota(jnp.int32, (G, block_m), 0)
    onehot = (ids[None, :] == gids).astype(jnp.float32)
    acc[...] += jnp.dot(onehot, hg_ref[...],
                        preferred_element_type=jnp.float32)
    cnt[...] += jnp.sum(onehot, axis=1, keepdims=True)

    @pl.when(i == nsteps - 1)
    def _():
        pooled = acc[...] / jnp.maximum(cnt[...], 1.0)
        o_ref[...] = jnp.dot(pooled, w_ref[...],
                             preferred_element_type=jnp.float32) + b_ref[...]


def pool_out(hg, batch, w, b, block_m=2000):
    nsteps = N // block_m
    return pl.pallas_call(
        functools.partial(_pool_out_body, block_m=block_m, nsteps=nsteps),
        grid=(nsteps,),
        in_specs=[
            pl.BlockSpec((block_m, F), lambda i: (i, 0)),
            pl.BlockSpec((1, 1, block_m), lambda i: (i, 0, 0)),
            pl.BlockSpec((F, 128), lambda i: (0, 0)),
            pl.BlockSpec((1, 128), lambda i: (0, 0)),
        ],
        out_specs=pl.BlockSpec((G, 128), lambda i: (0, 0)),
        out_shape=jax.ShapeDtypeStruct((G, 128), jnp.float32),
        scratch_shapes=[
            pltpu.VMEM((G, F), jnp.float32),
            pltpu.VMEM((G, 1), jnp.float32),
        ],
    )(hg, batch.astype(jnp.int32).reshape(nsteps, 1, block_m), w,
      b.reshape(1, 128))


NS_PAD = 102400
ZS_CH = NS_PAD // 32
ZS_NCHUNK = ZS_CH // 128
ZACC_R = 10240
ZPAD_IDX = 10016

_MESH_CACHE = []


def _sc_mesh():
    if not _MESH_CACHE:
        _MESH_CACHE.append(plsc.VectorSubcoreMesh(core_axis_name="c",
                                                  subcore_axis_name="s"))
    return _MESH_CACHE[0]


def _zs_body(h2_hbm, sb_hbm, out_hbm, idx_v, rows_v, zb_v, acc_sh, sem):
    c = lax.axis_index("c")
    s = lax.axis_index("s")
    wid = c * 16 + s
    base = wid * ZS_CH

    zb_v[...] = jnp.zeros_like(zb_v)
    @pl.loop(0, ZACC_R // 16 // 64)
    def _(j):
        pltpu.sync_copy(zb_v, acc_sh.at[pl.ds(s * (ZACC_R // 16) + j * 64, 64)])
    pltpu.async_copy(sb_hbm.at[wid], idx_v, sem).wait()
    plsc.subcore_barrier()

    @pl.loop(0, ZS_NCHUNK)
    def _(j):
        pltpu.async_copy(h2_hbm.at[pl.ds(base + j * 128, 128)],
                         rows_v, sem).wait()
        pltpu.sync_copy(rows_v, acc_sh.at[idx_v.at[j]], add=True)

    plsc.subcore_barrier()
    pltpu.sync_copy(acc_sh.at[pl.ds(s * 640, 640)],
                    out_hbm.at[c, pl.ds(s * 640, 640)])


def sc_zs(h2, sub_batch_pad):
    kern = pl.kernel(
        _zs_body,
        out_type=jax.ShapeDtypeStruct((2, ZACC_R, LAT), jnp.float32),
        mesh=_sc_mesh(),
        scratch_types=[
            pltpu.VMEM((ZS_NCHUNK, 128), jnp.int32),
            pltpu.VMEM((128, LAT), jnp.float32),
            pltpu.VMEM((64, LAT), jnp.float32),
            pltpu.VMEM_SHARED((ZACC_R, LAT), jnp.float32),
            pltpu.SemaphoreType.DMA,
        ],
    )
    return kern(h2, sub_batch_pad.reshape(32, ZS_NCHUNK, 128))


def _cnt_body(sb_hbm, out_hbm, idx_v, ones_v, zb_v, acc_sh, sem):
    c = lax.axis_index("c")
    s = lax.axis_index("s")
    wid = c * 16 + s

    zb_v[...] = jnp.zeros_like(zb_v)
    @pl.loop(0, ZACC_R // 16 // 64)
    def _(j):
        pltpu.sync_copy(zb_v, acc_sh.at[pl.ds(s * (ZACC_R // 16) + j * 64, 64)])
    ones_v[...] = jnp.ones_like(ones_v)
    pltpu.async_copy(sb_hbm.at[wid], idx_v, sem).wait()
    plsc.subcore_barrier()

    @pl.loop(0, ZS_NCHUNK)
    def _(j):
        pltpu.sync_copy(ones_v, acc_sh.at[idx_v.at[j]], add=True)

    plsc.subcore_barrier()
    pltpu.sync_copy(acc_sh.at[pl.ds(s * 640, 640)],
                    out_hbm.at[c, pl.ds(s * 640, 640)])


def sc_cnt(sub_batch_pad):
    kern = pl.kernel(
        _cnt_body,
        out_type=jax.ShapeDtypeStruct((2, ZACC_R, LAT), jnp.float32),
        mesh=_sc_mesh(),
        scratch_types=[
            pltpu.VMEM((ZS_NCHUNK, 128), jnp.int32),
            pltpu.VMEM((128, LAT), jnp.float32),
            pltpu.VMEM((64, LAT), jnp.float32),
            pltpu.VMEM_SHARED((ZACC_R, LAT), jnp.float32),
            pltpu.SemaphoreType.DMA,
        ],
    )
    return kern(sub_batch_pad.reshape(32, ZS_NCHUNK, 128))


ES_PAD = 425984
AG_CH = ES_PAD // 16
CH_E = 3328
AG_NCHUNK = AG_CH // CH_E
AG_R = 10000
AG_PASSES = 5
AG_C = 128
AG_THRESH = AG_C - 16


def _agg_body(h_hbm, s_hbm, d_hbm, agg_hbm, s_v, d_v, sbuf, dbuf, rows_v,
              zb_v, acc_sh, sem):
    c = lax.axis_index("c")
    s = lax.axis_index("s")
    wid = c * 16 + s
    lane = lax.iota(jnp.int32, 16)
    dummy_src = wid * 16 + lane
    dummy_dst = 10000 + s * 14 + lane
    zeros16 = jnp.zeros((16,), jnp.float32)

    def refill():
        @pl.loop(0, 8)
        def _(q):
            sbuf[0, pl.ds(q * 16, 16)] = dummy_src
            dbuf[0, pl.ds(q * 16, 16)] = dummy_dst

    def flush():
        pltpu.async_copy(h_hbm.at[sbuf.at[0]], rows_v, sem).wait()
        pltpu.sync_copy(rows_v, acc_sh.at[dbuf.at[0]], add=True)
        refill()

    @pl.loop(0, 64)
    def _(r):
        @pl.loop(0, 8)
        def _(q):
            zb_v[r, pl.ds(q * 16, 16)] = zeros16
    refill()

    for p in range(AG_PASSES):
        lo = c * (AG_PASSES * AG_R) + p * AG_R

        @pl.loop(0, ZACC_R // 16 // 64)
        def _(j):
            pltpu.sync_copy(zb_v,
                            acc_sh.at[pl.ds(s * (ZACC_R // 16) + j * 64, 64)])
        plsc.subcore_barrier()

        def chunk_body(ci, cnt0):
            pltpu.async_copy(
                s_hbm.at[pl.ds(s * AG_CH + ci * CH_E, CH_E)], s_v, sem).wait()
            pltpu.async_copy(
                d_hbm.at[pl.ds(s * AG_CH + ci * CH_E, CH_E)], d_v, sem).wait()

            def body(i, cnt):
                dv = d_v[pl.ds(i * 16, 16)]
                sv = s_v[pl.ds(i * 16, 16)]
                ldv = dv - lo
                m = (ldv >= 0) & (ldv < AG_R)
                pos = cnt + plsc.cumsum(m.astype(jnp.int32)) - 1
                plsc.store_scatter(sbuf, [pos - pos, pos], sv, mask=m)
                plsc.store_scatter(dbuf, [pos - pos, pos], ldv, mask=m)
                newcnt = cnt + jnp.max(plsc.all_reduce_population_count(m))
                do_flush = newcnt >= AG_THRESH

                @pl.when(do_flush)
                def _():
                    flush()

                return jnp.where(do_flush, 0, newcnt)

            return lax.fori_loop(0, CH_E // 16, body, cnt0)

        lax.fori_loop(0, AG_NCHUNK, chunk_body, jnp.int32(0))
        flush()
        plsc.subcore_barrier()

        @pl.when(s < 15)
        def _():
            pltpu.sync_copy(acc_sh.at[pl.ds(s * 640, 640)],
                            agg_hbm.at[pl.ds(lo + s * 640, 640)])

        @pl.when(s == 15)
        def _():
            pltpu.sync_copy(acc_sh.at[pl.ds(9600, 400)],
                            agg_hbm.at[pl.ds(lo + 9600, 400)])

        plsc.subcore_barrier()


def sc_agg(h, s_pad, d_pad):
    kern = pl.kernel(
        _agg_body,
        compiler_params=pltpu.CompilerParams(needs_layout_passes=False),
        out_type=jax.ShapeDtypeStruct((NS, LAT), jnp.float32),
        mesh=_sc_mesh(),
        scratch_types=[
            pltpu.VMEM((CH_E,), jnp.int32),
            pltpu.VMEM((CH_E,), jnp.int32),
            pltpu.VMEM((1, 128), jnp.int32),
            pltpu.VMEM((1, 128), jnp.int32),
            pltpu.VMEM((AG_C, LAT), jnp.float32),
            pltpu.VMEM((64, LAT), jnp.float32),
            pltpu.VMEM_SHARED((ZACC_R, LAT), jnp.float32),
            pltpu.SemaphoreType.DMA,
        ],
    )
    return kern(h, s_pad, d_pad)


def kernel(x, adj_t, pos, batch, sub_x, sub_adj_t, sub_batch, edge_index,
           batch_lengths, edge_batch, We1, be1, We2, be2, gamma, beta,
           Wg1, bg1, Wg2, bg2):
    h = mm_bias(sub_x, We1, be1, relu=True)
    s_pad = jnp.pad(sub_adj_t[0].astype(jnp.int32), (0, ES_PAD - ES))
    d_pad = jnp.pad(sub_adj_t[1].astype(jnp.int32), (0, ES_PAD - ES),
                    constant_values=2 ** 30)
    agg = sc_agg(h, s_pad, d_pad)
    h2 = mm_bias(agg, We2, be2, relu=True, out_rows=NS_PAD)
    sb_pad = jnp.pad(sub_batch.astype(jnp.int32), (0, NS_PAD - NS),
                     constant_values=ZPAD_IDX)
    zcnt = sc_zs(h2, sb_pad)
    cnts = sc_cnt(sb_pad)
    xz, sums, sq = bn_stats(x, zcnt, cnts)
    xn = bn_apply(xz, sums, sq, gamma, beta)
    es, ed = edge_index[0], edge_index[1]
    w = jnp.exp(-jnp.sum((pos[es] - pos[ed]) ** 2, axis=1))
    m = jax.ops.segment_sum(xn[es] * w[:, None], ed, num_segments=N)
    hg = mm_bias(m, Wg1, bg1, relu=True)
    return pool_out(hg, batch, Wg2, bg2)


# speedup vs baseline: 3.3657x; 2.1207x over previous
import functools

import jax
import jax.numpy as jnp
from jax import lax
from jax.experimental import pallas as pl
from jax.experimental.pallas import tpu as pltpu
from jax.experimental.pallas import tpu_sc as plsc

N, D, LAT, NS, ES, E, G = 10000, 128, 128, 100000, 400000, 160000, 64
F = D + LAT


def _mm_split_body(a_ref, w_ref, b_ref, o_ref):
    acc = jnp.dot(a_ref[0], w_ref[:128, :], preferred_element_type=jnp.float32)
    acc += jnp.dot(a_ref[1], w_ref[128:, :], preferred_element_type=jnp.float32)
    o_ref[...] = jnp.maximum(acc + b_ref[...], 0.0)


def mm_split(m2, w, b, block_m=2000):
    n = w.shape[1]
    return pl.pallas_call(
        _mm_split_body,
        grid=(N // block_m,),
        in_specs=[
            pl.BlockSpec((2, block_m, 128), lambda i: (0, i, 0)),
            pl.BlockSpec((F, n), lambda i: (0, 0)),
            pl.BlockSpec((1, n), lambda i: (0, 0)),
        ],
        out_specs=pl.BlockSpec((block_m, n), lambda i: (i, 0)),
        out_shape=jax.ShapeDtypeStruct((N, n), jnp.float32),
    )(m2, w, b.reshape(1, n))


def _mm_bias_body(a_ref, w_ref, b_ref, o_ref, *, relu):
    acc = jnp.dot(a_ref[...], w_ref[...], preferred_element_type=jnp.float32)
    acc = acc + b_ref[...]
    if relu:
        acc = jnp.maximum(acc, 0.0)
    o_ref[...] = acc


def mm_bias(a, w, b, relu, block_m=2000, out_rows=None):
    m, k = a.shape
    n = w.shape[1]
    assert m % block_m == 0
    return pl.pallas_call(
        functools.partial(_mm_bias_body, relu=relu),
        grid=(m // block_m,),
        in_specs=[
            pl.BlockSpec((block_m, k), lambda i: (i, 0)),
            pl.BlockSpec((k, n), lambda i: (0, 0)),
            pl.BlockSpec((1, n), lambda i: (0, 0)),
        ],
        out_specs=pl.BlockSpec((block_m, n), lambda i: (i, 0)),
        out_shape=jax.ShapeDtypeStruct((out_rows or m, n), jnp.float32),
    )(a, w, b.reshape(1, n))


def _bn_stats_body(x_ref, zc_ref, ct_ref, xz_ref, sums_ref, sq_ref,
                   acc_s, acc_q, *, nsteps):
    i = pl.program_id(0)

    @pl.when(i == 0)
    def _():
        acc_s[...] = jnp.zeros_like(acc_s)
        acc_q[...] = jnp.zeros_like(acc_q)

    zc = zc_ref[0] + zc_ref[1]
    ct = ct_ref[0, :, 0:1] + ct_ref[1, :, 0:1]
    z = zc / jnp.maximum(ct, 1.0)
    xz = jnp.concatenate((x_ref[...], z), axis=1)
    xz_ref[...] = xz
    acc_s[...] += jnp.sum(xz, axis=0, keepdims=True)
    acc_q[...] += jnp.sum(xz * xz, axis=0, keepdims=True)

    @pl.when(i == nsteps - 1)
    def _():
        sums_ref[...] = acc_s[...]
        sq_ref[...] = acc_q[...]


def bn_stats(x, zcnt, cnts, block_m=2000):
    nsteps = N // block_m
    return pl.pallas_call(
        functools.partial(_bn_stats_body, nsteps=nsteps),
        grid=(nsteps,),
        in_specs=[
            pl.BlockSpec((block_m, D), lambda i: (i, 0)),
            pl.BlockSpec((2, block_m, LAT), lambda i: (0, i, 0)),
            pl.BlockSpec((2, block_m, LAT), lambda i: (0, i, 0)),
        ],
        out_specs=[
            pl.BlockSpec((block_m, F), lambda i: (i, 0)),
            pl.BlockSpec((1, F), lambda i: (0, 0)),
            pl.BlockSpec((1, F), lambda i: (0, 0)),
        ],
        out_shape=[
            jax.ShapeDtypeStruct((N, F), jnp.float32),
            jax.ShapeDtypeStruct((1, F), jnp.float32),
            jax.ShapeDtypeStruct((1, F), jnp.float32),
        ],
        scratch_shapes=[
            pltpu.VMEM((1, F), jnp.float32),
            pltpu.VMEM((1, F), jnp.float32),
        ],
    )(x, zcnt, cnts)


def _bn_apply_body(xz_ref, s_ref, q_ref, g_ref, b_ref, o_ref):
    mu = s_ref[...] / N
    var = q_ref[...] / N - mu * mu
    rstd = jax.lax.rsqrt(var + 1e-5)
    xn = (xz_ref[...] - mu) * rstd * g_ref[...] + b_ref[...]
    o_ref[0] = xn[:, :128]
    o_ref[1] = xn[:, 128:]


def bn_apply(xz, sums, sq, gamma, beta, block_m=2000):
    return pl.pallas_call(
        _bn_apply_body,
        grid=(N // block_m,),
        in_specs=[
            pl.BlockSpec((block_m, F), lambda i: (i, 0)),
            pl.BlockSpec((1, F), lambda i: (0, 0)),
            pl.BlockSpec((1, F), lambda i: (0, 0)),
            pl.BlockSpec((1, F), lambda i: (0, 0)),
            pl.BlockSpec((1, F), lambda i: (0, 0)),
        ],
        out_specs=pl.BlockSpec((2, block_m, 128), lambda i: (0, i, 0)),
        out_shape=jax.ShapeDtypeStruct((2, N, 128), jnp.float32),
    )(xz, sums, sq, gamma.reshape(1, F), beta.reshape(1, F))


def _pool_out_body(hg_ref, batch_ref, w_ref, b_ref, o_ref, acc, cnt,
                   *, block_m, nsteps):
    i = pl.program_id(0)

    @pl.when(i == 0)
    def _():
        acc[...] = jnp.zeros_like(acc)
        cnt[...] = jnp.zeros_like(cnt)

    ids = batch_ref[0, 0, :]
    gids = jax.lax.broadcasted_iota(jnp.int32, (G, block_m), 0)
    onehot = (ids[None, :] == gids).astype(jnp.float32)
    acc[...] += jnp.dot(onehot, hg_ref[...],
                        preferred_element_type=jnp.float32)
    cnt[...] += jnp.sum(onehot, axis=1, keepdims=True)

    @pl.when(i == nsteps - 1)
    def _():
        pooled = acc[...] / jnp.maximum(cnt[...], 1.0)
        o_ref[...] = jnp.dot(pooled, w_ref[...],
                             preferred_element_type=jnp.float32) + b_ref[...]


def pool_out(hg, batch, w, b, block_m=2000):
    nsteps = N // block_m
    return pl.pallas_call(
        functools.partial(_pool_out_body, block_m=block_m, nsteps=nsteps),
        grid=(nsteps,),
        in_specs=[
            pl.BlockSpec((block_m, F), lambda i: (i, 0)),
            pl.BlockSpec((1, 1, block_m), lambda i: (i, 0, 0)),
            pl.BlockSpec((F, 128), lambda i: (0, 0)),
            pl.BlockSpec((1, 128), lambda i: (0, 0)),
        ],
        out_specs=pl.BlockSpec((G, 128), lambda i: (0, 0)),
        out_shape=jax.ShapeDtypeStruct((G, 128), jnp.float32),
        scratch_shapes=[
            pltpu.VMEM((G, F), jnp.float32),
            pltpu.VMEM((G, 1), jnp.float32),
        ],
    )(hg, batch.astype(jnp.int32).reshape(nsteps, 1, block_m), w,
      b.reshape(1, 128))


NS_PAD = 102400
ZS_CH = NS_PAD // 32
ZS_NCHUNK = ZS_CH // 128
ZACC_R = 10240
ZPAD_IDX = 10016

_MESH_CACHE = []


def _sc_mesh():
    if not _MESH_CACHE:
        _MESH_CACHE.append(plsc.VectorSubcoreMesh(core_axis_name="c",
                                                  subcore_axis_name="s"))
    return _MESH_CACHE[0]


def _zs_body(h2_hbm, sb_hbm, out_hbm, idx_v, rows_v, zb_v, acc_sh, sem):
    c = lax.axis_index("c")
    s = lax.axis_index("s")
    wid = c * 16 + s
    base = wid * ZS_CH

    zb_v[...] = jnp.zeros_like(zb_v)
    @pl.loop(0, ZACC_R // 16 // 64)
    def _(j):
        pltpu.sync_copy(zb_v, acc_sh.at[pl.ds(s * (ZACC_R // 16) + j * 64, 64)])
    pltpu.async_copy(sb_hbm.at[wid], idx_v, sem).wait()
    plsc.subcore_barrier()

    @pl.loop(0, ZS_NCHUNK)
    def _(j):
        pltpu.async_copy(h2_hbm.at[pl.ds(base + j * 128, 128)],
                         rows_v, sem).wait()
        pltpu.sync_copy(rows_v, acc_sh.at[idx_v.at[j]], add=True)

    plsc.subcore_barrier()
    pltpu.sync_copy(acc_sh.at[pl.ds(s * 640, 640)],
                    out_hbm.at[c, pl.ds(s * 640, 640)])


def sc_zs(h2, sub_batch_pad):
    kern = pl.kernel(
        _zs_body,
        out_type=jax.ShapeDtypeStruct((2, ZACC_R, LAT), jnp.float32),
        mesh=_sc_mesh(),
        scratch_types=[
            pltpu.VMEM((ZS_NCHUNK, 128), jnp.int32),
            pltpu.VMEM((128, LAT), jnp.float32),
            pltpu.VMEM((64, LAT), jnp.float32),
            pltpu.VMEM_SHARED((ZACC_R, LAT), jnp.float32),
            pltpu.SemaphoreType.DMA,
        ],
    )
    return kern(h2, sub_batch_pad.reshape(32, ZS_NCHUNK, 128))


def _cnt_body(sb_hbm, out_hbm, idx_v, ones_v, zb_v, acc_sh, sem):
    c = lax.axis_index("c")
    s = lax.axis_index("s")
    wid = c * 16 + s

    zb_v[...] = jnp.zeros_like(zb_v)
    @pl.loop(0, ZACC_R // 16 // 64)
    def _(j):
        pltpu.sync_copy(zb_v, acc_sh.at[pl.ds(s * (ZACC_R // 16) + j * 64, 64)])
    ones_v[...] = jnp.ones_like(ones_v)
    pltpu.async_copy(sb_hbm.at[wid], idx_v, sem).wait()
    plsc.subcore_barrier()

    @pl.loop(0, ZS_NCHUNK)
    def _(j):
        pltpu.sync_copy(ones_v, acc_sh.at[idx_v.at[j]], add=True)

    plsc.subcore_barrier()
    pltpu.sync_copy(acc_sh.at[pl.ds(s * 640, 640)],
                    out_hbm.at[c, pl.ds(s * 640, 640)])


def sc_cnt(sub_batch_pad):
    kern = pl.kernel(
        _cnt_body,
        out_type=jax.ShapeDtypeStruct((2, ZACC_R, LAT), jnp.float32),
        mesh=_sc_mesh(),
        scratch_types=[
            pltpu.VMEM((ZS_NCHUNK, 128), jnp.int32),
            pltpu.VMEM((128, LAT), jnp.float32),
            pltpu.VMEM((64, LAT), jnp.float32),
            pltpu.VMEM_SHARED((ZACC_R, LAT), jnp.float32),
            pltpu.SemaphoreType.DMA,
        ],
    )
    return kern(sub_batch_pad.reshape(32, ZS_NCHUNK, 128))


ES_PAD = 425984
AG_CH = ES_PAD // 16
CH_E = 3328
AG_NCHUNK = AG_CH // CH_E
AG_R = 10000
AG_PASSES = 5
AG_C = 128
AG_THRESH = AG_C - 16


def _agg_body(h_hbm, s_hbm, d_hbm, agg_hbm, s_v, d_v, sbuf, dbuf, rows_v,
              zb_v, acc_sh, sem):
    c = lax.axis_index("c")
    s = lax.axis_index("s")
    wid = c * 16 + s
    lane = lax.iota(jnp.int32, 16)
    dummy_src = wid * 16 + lane
    dummy_dst = 10000 + s * 14 + lane
    zeros16 = jnp.zeros((16,), jnp.float32)

    def refill():
        @pl.loop(0, 8)
        def _(q):
            sbuf[0, pl.ds(q * 16, 16)] = dummy_src
            dbuf[0, pl.ds(q * 16, 16)] = dummy_dst

    def flush():
        pltpu.async_copy(h_hbm.at[sbuf.at[0]], rows_v, sem).wait()
        pltpu.sync_copy(rows_v, acc_sh.at[dbuf.at[0]], add=True)
        refill()

    @pl.loop(0, 64)
    def _(r):
        @pl.loop(0, 8)
        def _(q):
            zb_v[r, pl.ds(q * 16, 16)] = zeros16
    refill()

    for p in range(AG_PASSES):
        lo = c * (AG_PASSES * AG_R) + p * AG_R

        @pl.loop(0, ZACC_R // 16 // 64)
        def _(j):
            pltpu.sync_copy(zb_v,
                            acc_sh.at[pl.ds(s * (ZACC_R // 16) + j * 64, 64)])
        plsc.subcore_barrier()

        def chunk_body(ci, cnt0):
            pltpu.async_copy(
                s_hbm.at[pl.ds(s * AG_CH + ci * CH_E, CH_E)], s_v, sem).wait()
            pltpu.async_copy(
                d_hbm.at[pl.ds(s * AG_CH + ci * CH_E, CH_E)], d_v, sem).wait()

            def body(i, cnt):
                dv = d_v[pl.ds(i * 16, 16)]
                sv = s_v[pl.ds(i * 16, 16)]
                ldv = dv - lo
                m = (ldv >= 0) & (ldv < AG_R)
                pos = cnt + plsc.cumsum(m.astype(jnp.int32)) - 1
                plsc.store_scatter(sbuf, [pos - pos, pos], sv, mask=m)
                plsc.store_scatter(dbuf, [pos - pos, pos], ldv, mask=m)
                newcnt = cnt + jnp.max(plsc.all_reduce_population_count(m))
                do_flush = newcnt >= AG_THRESH

                @pl.when(do_flush)
                def _():
                    flush()

                return jnp.where(do_flush, 0, newcnt)

            return lax.fori_loop(0, CH_E // 16, body, cnt0)

        lax.fori_loop(0, AG_NCHUNK, chunk_body, jnp.int32(0))
        flush()
        plsc.subcore_barrier()

        @pl.when(s < 15)
        def _():
            pltpu.sync_copy(acc_sh.at[pl.ds(s * 640, 640)],
                            agg_hbm.at[pl.ds(lo + s * 640, 640)])

        @pl.when(s == 15)
        def _():
            pltpu.sync_copy(acc_sh.at[pl.ds(9600, 400)],
                            agg_hbm.at[pl.ds(lo + 9600, 400)])

        plsc.subcore_barrier()


def sc_agg(h, s_pad, d_pad):
    kern = pl.kernel(
        _agg_body,
        compiler_params=pltpu.CompilerParams(needs_layout_passes=False),
        out_type=jax.ShapeDtypeStruct((NS, LAT), jnp.float32),
        mesh=_sc_mesh(),
        scratch_types=[
            pltpu.VMEM((CH_E,), jnp.int32),
            pltpu.VMEM((CH_E,), jnp.int32),
            pltpu.VMEM((1, 128), jnp.int32),
            pltpu.VMEM((1, 128), jnp.int32),
            pltpu.VMEM((AG_C, LAT), jnp.float32),
            pltpu.VMEM((64, LAT), jnp.float32),
            pltpu.VMEM_SHARED((ZACC_R, LAT), jnp.float32),
            pltpu.SemaphoreType.DMA,
        ],
    )
    return kern(h, s_pad, d_pad)


E_PAD = 163840
W_CH = E_PAD // 32
M_CH = E_PAD // 16
M_SUB = 1024


def _w_body(px_h, py_h, pz_h, es_h, ed_h, w_h, px_v, py_v, pz_v,
            es_v, ed_v, w_v, sem):
    c = lax.axis_index("c")
    s = lax.axis_index("s")
    wid = c * 16 + s
    base = wid * W_CH
    pltpu.async_copy(px_h, px_v, sem).wait()
    pltpu.async_copy(py_h, py_v, sem).wait()
    pltpu.async_copy(pz_h, pz_v, sem).wait()
    pltpu.async_copy(es_h.at[pl.ds(base, W_CH)], es_v, sem).wait()
    pltpu.async_copy(ed_h.at[pl.ds(base, W_CH)], ed_v, sem).wait()

    @pl.loop(0, W_CH // 16)
    def _(i):
        esv = es_v[pl.ds(i * 16, 16)]
        edv = ed_v[pl.ds(i * 16, 16)]
        dx = plsc.load_gather(px_v, [esv]) - plsc.load_gather(px_v, [edv])
        dy = plsc.load_gather(py_v, [esv]) - plsc.load_gather(py_v, [edv])
        dz = plsc.load_gather(pz_v, [esv]) - plsc.load_gather(pz_v, [edv])
        w_v[pl.ds(i * 16, 16)] = jnp.exp(-(dx * dx + dy * dy + dz * dz))

    pltpu.sync_copy(w_v, w_h.at[pl.ds(base, W_CH)])


def sc_w(px, py, pz, es_w, ed_w):
    kern = pl.kernel(
        _w_body,
        compiler_params=pltpu.CompilerParams(needs_layout_passes=False),
        out_type=jax.ShapeDtypeStruct((E_PAD,), jnp.float32),
        mesh=_sc_mesh(),
        scratch_types=[
            pltpu.VMEM((N,), jnp.float32),
            pltpu.VMEM((N,), jnp.float32),
            pltpu.VMEM((N,), jnp.float32),
            pltpu.VMEM((W_CH,), jnp.int32),
            pltpu.VMEM((W_CH,), jnp.int32),
            pltpu.VMEM((W_CH,), jnp.float32),
            pltpu.SemaphoreType.DMA,
        ],
    )
    return kern(px, py, pz, es_w, ed_w)


def _m_body(xn_h, es_h, ed_h, w_h, m_h, es_c, ed_c, w_c, rows_v, zb_v,
            acc_sh, sem):
    c = lax.axis_index("c")
    s = lax.axis_index("s")
    zeros16 = jnp.zeros((16,), jnp.float32)

    @pl.loop(0, 32)
    def _(r):
        @pl.loop(0, 8)
        def _(q):
            zb_v[r, pl.ds(q * 16, 16)] = zeros16

    @pl.loop(0, ZACC_R // 16 // 32)
    def _(j):
        pltpu.sync_copy(zb_v, acc_sh.at[pl.ds(s * (ZACC_R // 16) + j * 32, 32)])
    plsc.subcore_barrier()

    @pl.loop(0, M_CH // M_SUB)
    def _(q):
        row0 = s * (M_CH // 128) + q * (M_SUB // 128)
        pltpu.async_copy(es_h.at[pl.ds(row0, M_SUB // 128)], es_c, sem).wait()
        pltpu.async_copy(ed_h.at[pl.ds(row0, M_SUB // 128)], ed_c, sem).wait()
        pltpu.async_copy(w_h.at[pl.ds(s * M_CH + q * M_SUB, M_SUB)],
                         w_c, sem).wait()

        @pl.loop(0, M_SUB // 128)
        def _(j):
            pltpu.async_copy(xn_h.at[c].at[es_c.at[j]], rows_v, sem).wait()

            @pl.loop(0, 128)
            def _(r):
                wsp = plsc.load_gather(w_c, [jnp.full((16,), j * 128 + r,
                                                      jnp.int32)])

                @pl.loop(0, 8)
                def _(k):
                    rows_v[r, pl.ds(k * 16, 16)] = (
                        rows_v[r, pl.ds(k * 16, 16)] * wsp)

            pltpu.sync_copy(rows_v, acc_sh.at[ed_c.at[j]], add=True)

    plsc.subcore_barrier()

    @pl.when(s < 15)
    def _():
        pltpu.sync_copy(acc_sh.at[pl.ds(s * 640, 640)],
                        m_h.at[c, pl.ds(s * 640, 640)])

    @pl.when(s == 15)
    def _():
        pltpu.sync_copy(acc_sh.at[pl.ds(9600, 640)],
                        m_h.at[c, pl.ds(9600, 640)])

    plsc.subcore_barrier()


def sc_m(xn2, es2d, ed2d, w):
    kern = pl.kernel(
        _m_body,
        compiler_params=pltpu.CompilerParams(needs_layout_passes=False),
        out_type=jax.ShapeDtypeStruct((2, ZACC_R, 128), jnp.float32),
        mesh=_sc_mesh(),
        scratch_types=[
            pltpu.VMEM((M_SUB // 128, 128), jnp.int32),
            pltpu.VMEM((M_SUB // 128, 128), jnp.int32),
            pltpu.VMEM((M_SUB,), jnp.float32),
            pltpu.VMEM((128, 128), jnp.float32),
            pltpu.VMEM((32, 128), jnp.float32),
            pltpu.VMEM_SHARED((ZACC_R, 128), jnp.float32),
            pltpu.SemaphoreType.DMA,
        ],
    )
    return kern(xn2, es2d, ed2d, w)


def kernel(x, adj_t, pos, batch, sub_x, sub_adj_t, sub_batch, edge_index,
           batch_lengths, edge_batch, We1, be1, We2, be2, gamma, beta,
           Wg1, bg1, Wg2, bg2):
    h = mm_bias(sub_x, We1, be1, relu=True)
    s_pad = jnp.pad(sub_adj_t[0].astype(jnp.int32), (0, ES_PAD - ES))
    d_pad = jnp.pad(sub_adj_t[1].astype(jnp.int32), (0, ES_PAD - ES),
                    constant_values=2 ** 30)
    agg = sc_agg(h, s_pad, d_pad)
    h2 = mm_bias(agg, We2, be2, relu=True, out_rows=NS_PAD)
    sb_pad = jnp.pad(sub_batch.astype(jnp.int32), (0, NS_PAD - NS),
                     constant_values=ZPAD_IDX)
    zcnt = sc_zs(h2, sb_pad)
    cnts = sc_cnt(sb_pad)
    xz, sums, sq = bn_stats(x, zcnt, cnts)
    xn2 = bn_apply(xz, sums, sq, gamma, beta)
    es = edge_index[0].astype(jnp.int32)
    ed = edge_index[1].astype(jnp.int32)
    es_w = jnp.pad(es, (0, E_PAD - E))
    ed_w = jnp.pad(ed, (0, E_PAD - E))
    w = sc_w(pos[:, 0], pos[:, 1], pos[:, 2], es_w, ed_w)
    ed_m = jnp.pad(ed, (0, E_PAD - E), constant_values=ZPAD_IDX)
    m2 = sc_m(xn2, es_w.reshape(E_PAD // 128, 128),
              ed_m.reshape(E_PAD // 128, 128), w)
    hg = mm_split(m2, Wg1, bg1)
    return pool_out(hg, batch, Wg2, bg2)


# speedup vs baseline: 3.5132x; 1.0438x over previous
import functools

import jax
import jax.numpy as jnp
from jax import lax
from jax.experimental import pallas as pl
from jax.experimental.pallas import tpu as pltpu
from jax.experimental.pallas import tpu_sc as plsc

N, D, LAT, NS, ES, E, G = 10000, 128, 128, 100000, 400000, 160000, 64
F = D + LAT


def _mm_split_body(a_ref, w_ref, b_ref, o_ref):
    acc = jnp.dot(a_ref[0], w_ref[:128, :], preferred_element_type=jnp.float32)
    acc += jnp.dot(a_ref[1], w_ref[128:, :], preferred_element_type=jnp.float32)
    o_ref[...] = jnp.maximum(acc + b_ref[...], 0.0)


def mm_split(m2, w, b, block_m=2000):
    n = w.shape[1]
    return pl.pallas_call(
        _mm_split_body,
        grid=(N // block_m,),
        in_specs=[
            pl.BlockSpec((2, block_m, 128), lambda i: (0, i, 0)),
            pl.BlockSpec((F, n), lambda i: (0, 0)),
            pl.BlockSpec((1, n), lambda i: (0, 0)),
        ],
        out_specs=pl.BlockSpec((block_m, n), lambda i: (i, 0)),
        out_shape=jax.ShapeDtypeStruct((N, n), jnp.float32),
    )(m2, w, b.reshape(1, n))


def _mm_bias_body(a_ref, w_ref, b_ref, o_ref, *, relu):
    acc = jnp.dot(a_ref[...], w_ref[...], preferred_element_type=jnp.float32)
    acc = acc + b_ref[...]
    if relu:
        acc = jnp.maximum(acc, 0.0)
    o_ref[...] = acc


def mm_bias(a, w, b, relu, block_m=2000, out_rows=None):
    m, k = a.shape
    n = w.shape[1]
    assert m % block_m == 0
    return pl.pallas_call(
        functools.partial(_mm_bias_body, relu=relu),
        grid=(m // block_m,),
        in_specs=[
            pl.BlockSpec((block_m, k), lambda i: (i, 0)),
            pl.BlockSpec((k, n), lambda i: (0, 0)),
            pl.BlockSpec((1, n), lambda i: (0, 0)),
        ],
        out_specs=pl.BlockSpec((block_m, n), lambda i: (i, 0)),
        out_shape=jax.ShapeDtypeStruct((out_rows or m, n), jnp.float32),
    )(a, w, b.reshape(1, n))


def _bn_stats_body(x_ref, zc_ref, ct_ref, xz_ref, sums_ref, sq_ref,
                   acc_s, acc_q, *, nsteps):
    i = pl.program_id(0)

    @pl.when(i == 0)
    def _():
        acc_s[...] = jnp.zeros_like(acc_s)
        acc_q[...] = jnp.zeros_like(acc_q)

    zc = zc_ref[0] + zc_ref[1]
    ct = ct_ref[0, :, 0:1] + ct_ref[1, :, 0:1]
    z = zc / jnp.maximum(ct, 1.0)
    xz = jnp.concatenate((x_ref[...], z), axis=1)
    xz_ref[...] = xz
    acc_s[...] += jnp.sum(xz, axis=0, keepdims=True)
    acc_q[...] += jnp.sum(xz * xz, axis=0, keepdims=True)

    @pl.when(i == nsteps - 1)
    def _():
        sums_ref[...] = acc_s[...]
        sq_ref[...] = acc_q[...]


def bn_stats(x, zcnt, cnts, block_m=2000):
    nsteps = N // block_m
    return pl.pallas_call(
        functools.partial(_bn_stats_body, nsteps=nsteps),
        grid=(nsteps,),
        in_specs=[
            pl.BlockSpec((block_m, D), lambda i: (i, 0)),
            pl.BlockSpec((2, block_m, LAT), lambda i: (0, i, 0)),
            pl.BlockSpec((2, block_m, LAT), lambda i: (0, i, 0)),
        ],
        out_specs=[
            pl.BlockSpec((block_m, F), lambda i: (i, 0)),
            pl.BlockSpec((1, F), lambda i: (0, 0)),
            pl.BlockSpec((1, F), lambda i: (0, 0)),
        ],
        out_shape=[
            jax.ShapeDtypeStruct((N, F), jnp.float32),
            jax.ShapeDtypeStruct((1, F), jnp.float32),
            jax.ShapeDtypeStruct((1, F), jnp.float32),
        ],
        scratch_shapes=[
            pltpu.VMEM((1, F), jnp.float32),
            pltpu.VMEM((1, F), jnp.float32),
        ],
    )(x, zcnt, cnts)


def _bn_apply_body(xz_ref, s_ref, q_ref, g_ref, b_ref, o_ref):
    mu = s_ref[...] / N
    var = q_ref[...] / N - mu * mu
    rstd = jax.lax.rsqrt(var + 1e-5)
    xn = (xz_ref[...] - mu) * rstd * g_ref[...] + b_ref[...]
    o_ref[0] = xn[:, :128]
    o_ref[1] = xn[:, 128:]


def bn_apply(xz, sums, sq, gamma, beta, block_m=2000):
    return pl.pallas_call(
        _bn_apply_body,
        grid=(N // block_m,),
        in_specs=[
            pl.BlockSpec((block_m, F), lambda i: (i, 0)),
            pl.BlockSpec((1, F), lambda i: (0, 0)),
            pl.BlockSpec((1, F), lambda i: (0, 0)),
            pl.BlockSpec((1, F), lambda i: (0, 0)),
            pl.BlockSpec((1, F), lambda i: (0, 0)),
        ],
        out_specs=pl.BlockSpec((2, block_m, 128), lambda i: (0, i, 0)),
        out_shape=jax.ShapeDtypeStruct((2, N, 128), jnp.float32),
    )(xz, sums, sq, gamma.reshape(1, F), beta.reshape(1, F))


def _pool_out_body(hg_ref, batch_ref, w_ref, b_ref, o_ref, acc, cnt,
                   *, block_m, nsteps):
    i = pl.program_id(0)

    @pl.when(i == 0)
    def _():
        acc[...] = jnp.zeros_like(acc)
        cnt[...] = jnp.zeros_like(cnt)

    ids = batch_ref[0, 0, :]
    gids = jax.lax.broadcasted_iota(jnp.int32, (G, block_m), 0)
    onehot = (ids[None, :] == gids).astype(jnp.float32)
    acc[...] += jnp.dot(onehot, hg_ref[...],
                        preferred_element_type=jnp.float32)
    cnt[...] += jnp.sum(onehot, axis=1, keepdims=True)

    @pl.when(i == nsteps - 1)
    def _():
        pooled = acc[...] / jnp.maximum(cnt[...], 1.0)
        o_ref[...] = jnp.dot(pooled, w_ref[...],
                             preferred_element_type=jnp.float32) + b_ref[...]


def pool_out(hg, batch, w, b, block_m=2000):
    nsteps = N // block_m
    return pl.pallas_call(
        functools.partial(_pool_out_body, block_m=block_m, nsteps=nsteps),
        grid=(nsteps,),
        in_specs=[
            pl.BlockSpec((block_m, F), lambda i: (i, 0)),
            pl.BlockSpec((1, 1, block_m), lambda i: (i, 0, 0)),
            pl.BlockSpec((F, 128), lambda i: (0, 0)),
            pl.BlockSpec((1, 128), lambda i: (0, 0)),
        ],
        out_specs=pl.BlockSpec((G, 128), lambda i: (0, 0)),
        out_shape=jax.ShapeDtypeStruct((G, 128), jnp.float32),
        scratch_shapes=[
            pltpu.VMEM((G, F), jnp.float32),
            pltpu.VMEM((G, 1), jnp.float32),
        ],
    )(hg, batch.astype(jnp.int32).reshape(nsteps, 1, block_m), w,
      b.reshape(1, 128))


NS_PAD = 102400
ZS_CH = NS_PAD // 32
ZS_NCHUNK = ZS_CH // 128
ZACC_R = 10240
ZPAD_IDX = 10016

_MESH_CACHE = []


def _sc_mesh():
    if not _MESH_CACHE:
        _MESH_CACHE.append(plsc.VectorSubcoreMesh(core_axis_name="c",
                                                  subcore_axis_name="s"))
    return _MESH_CACHE[0]


def _zs_body(h2_hbm, sb_hbm, out_hbm, idx_v, rows_v, zb_v, acc_sh, sem):
    c = lax.axis_index("c")
    s = lax.axis_index("s")
    wid = c * 16 + s
    base = wid * ZS_CH

    zb_v[...] = jnp.zeros_like(zb_v)
    @pl.loop(0, ZACC_R // 16 // 64)
    def _(j):
        pltpu.sync_copy(zb_v, acc_sh.at[pl.ds(s * (ZACC_R // 16) + j * 64, 64)])
    pltpu.async_copy(sb_hbm.at[wid], idx_v, sem).wait()
    plsc.subcore_barrier()

    @pl.loop(0, ZS_NCHUNK)
    def _(j):
        pltpu.async_copy(h2_hbm.at[pl.ds(base + j * 128, 128)],
                         rows_v, sem).wait()
        pltpu.sync_copy(rows_v, acc_sh.at[idx_v.at[j]], add=True)

    plsc.subcore_barrier()
    pltpu.sync_copy(acc_sh.at[pl.ds(s * 640, 640)],
                    out_hbm.at[c, pl.ds(s * 640, 640)])


def sc_zs(h2, sub_batch_pad):
    kern = pl.kernel(
        _zs_body,
        out_type=jax.ShapeDtypeStruct((2, ZACC_R, LAT), jnp.float32),
        mesh=_sc_mesh(),
        scratch_types=[
            pltpu.VMEM((ZS_NCHUNK, 128), jnp.int32),
            pltpu.VMEM((128, LAT), jnp.float32),
            pltpu.VMEM((64, LAT), jnp.float32),
            pltpu.VMEM_SHARED((ZACC_R, LAT), jnp.float32),
            pltpu.SemaphoreType.DMA,
        ],
    )
    return kern(h2, sub_batch_pad.reshape(32, ZS_NCHUNK, 128))


def _cnt_body(sb_hbm, out_hbm, idx_v, ones_v, zb_v, acc_sh, sem):
    c = lax.axis_index("c")
    s = lax.axis_index("s")
    wid = c * 16 + s

    zb_v[...] = jnp.zeros_like(zb_v)
    @pl.loop(0, ZACC_R // 16 // 64)
    def _(j):
        pltpu.sync_copy(zb_v, acc_sh.at[pl.ds(s * (ZACC_R // 16) + j * 64, 64)])
    ones_v[...] = jnp.ones_like(ones_v)
    pltpu.async_copy(sb_hbm.at[wid], idx_v, sem).wait()
    plsc.subcore_barrier()

    @pl.loop(0, ZS_NCHUNK)
    def _(j):
        pltpu.sync_copy(ones_v, acc_sh.at[idx_v.at[j]], add=True)

    plsc.subcore_barrier()
    pltpu.sync_copy(acc_sh.at[pl.ds(s * 640, 640)],
                    out_hbm.at[c, pl.ds(s * 640, 640)])


def sc_cnt(sub_batch_pad):
    kern = pl.kernel(
        _cnt_body,
        out_type=jax.ShapeDtypeStruct((2, ZACC_R, LAT), jnp.float32),
        mesh=_sc_mesh(),
        scratch_types=[
            pltpu.VMEM((ZS_NCHUNK, 128), jnp.int32),
            pltpu.VMEM((128, LAT), jnp.float32),
            pltpu.VMEM((64, LAT), jnp.float32),
            pltpu.VMEM_SHARED((ZACC_R, LAT), jnp.float32),
            pltpu.SemaphoreType.DMA,
        ],
    )
    return kern(sub_batch_pad.reshape(32, ZS_NCHUNK, 128))


ES_PAD = 425984
AG_CH = ES_PAD // 16
CH_E = 2048
AG_NCHUNK = AG_CH // CH_E
AG_R = 10000
AG_PASSES = 5
AG_C = 128
AG_THRESH = AG_C - 16


def _agg_body(h_hbm, s_hbm, d_hbm, agg_hbm, s_v0, d_v0, s_v1, d_v1,
              sbufc, dbufc, sbuf0, dbuf0, sbuf1, dbuf1, rows0, rows1,
              zb_v, acc_sh, sem_e0, sem_e1, sem_g):
    c = lax.axis_index("c")
    s = lax.axis_index("s")
    wid = c * 16 + s
    lane = lax.iota(jnp.int32, 16)
    dummy_src = wid * 16 + lane
    dummy_dst = 10000 + s * 14 + lane
    zeros16 = jnp.zeros((16,), jnp.float32)
    ebase = s * AG_CH

    def refill_c():
        @pl.loop(0, 8)
        def _(q):
            sbufc[0, pl.ds(q * 16, 16)] = dummy_src
            dbufc[0, pl.ds(q * 16, 16)] = dummy_dst

    def stage_to(sb, db):
        @pl.loop(0, 8)
        def _(q):
            sb[0, pl.ds(q * 16, 16)] = sbufc[0, pl.ds(q * 16, 16)]
            db[0, pl.ds(q * 16, 16)] = dbufc[0, pl.ds(q * 16, 16)]
        refill_c()

    def wait_scatter(sb, db, rows):
        pltpu.make_async_copy(h_hbm.at[sb.at[0]], rows, sem_g).wait()
        pltpu.sync_copy(rows, acc_sh.at[db.at[0]], add=True)

    @pl.loop(0, 32)
    def _(r):
        @pl.loop(0, 8)
        def _(q):
            zb_v[r, pl.ds(q * 16, 16)] = zeros16
    refill_c()

    for p in range(AG_PASSES):
        lo = c * (AG_PASSES * AG_R) + p * AG_R

        @pl.loop(0, ZACC_R // 16 // 32)
        def _(j):
            pltpu.sync_copy(zb_v,
                            acc_sh.at[pl.ds(s * (ZACC_R // 16) + j * 32, 32)])
        plsc.subcore_barrier()

        pltpu.async_copy(s_hbm.at[pl.ds(ebase, CH_E)], s_v0, sem_e0)
        pltpu.async_copy(d_hbm.at[pl.ds(ebase, CH_E)], d_v0, sem_e0)

        def scan_chunk(ci, carry, sv_ref, dv_ref, nsv_ref, ndv_ref,
                       sem_cur, sem_nxt):
            pltpu.make_async_copy(s_hbm.at[pl.ds(ebase + ci * CH_E, CH_E)],
                                  sv_ref, sem_cur).wait()
            pltpu.make_async_copy(d_hbm.at[pl.ds(ebase + ci * CH_E, CH_E)],
                                  dv_ref, sem_cur).wait()

            @pl.when(ci < AG_NCHUNK - 1)
            def _():
                pltpu.async_copy(
                    s_hbm.at[pl.ds(ebase + (ci + 1) * CH_E, CH_E)],
                    nsv_ref, sem_nxt)
                pltpu.async_copy(
                    d_hbm.at[pl.ds(ebase + (ci + 1) * CH_E, CH_E)],
                    ndv_ref, sem_nxt)

            def body(i, cf):
                cnt, f = cf
                dv = dv_ref[pl.ds(i * 16, 16)]
                sv = sv_ref[pl.ds(i * 16, 16)]
                ldv = dv - lo
                m = (ldv >= 0) & (ldv < AG_R)
                pos = cnt + plsc.cumsum(m.astype(jnp.int32)) - 1
                plsc.store_scatter(sbufc, [pos - pos, pos], sv, mask=m)
                plsc.store_scatter(dbufc, [pos - pos, pos], ldv, mask=m)
                newcnt = cnt + jnp.max(plsc.all_reduce_population_count(m))
                do_flush = newcnt >= AG_THRESH
                cp = lax.bitwise_and(f, 1)

                @pl.when(do_flush & (cp == 0))
                def _():
                    @pl.when(f > 0)
                    def _():
                        wait_scatter(sbuf1, dbuf1, rows1)
                    stage_to(sbuf0, dbuf0)
                    pltpu.async_copy(h_hbm.at[sbuf0.at[0]], rows0, sem_g)

                @pl.when(do_flush & (cp == 1))
                def _():
                    wait_scatter(sbuf0, dbuf0, rows0)
                    stage_to(sbuf1, dbuf1)
                    pltpu.async_copy(h_hbm.at[sbuf1.at[0]], rows1, sem_g)

                return (jnp.where(do_flush, 0, newcnt),
                        jnp.where(do_flush, f + 1, f))

            return lax.fori_loop(0, CH_E // 16, body, carry)

        def chunk_body(ci, carry):
            return lax.cond(
                lax.bitwise_and(ci, 1) == 0,
                lambda cr: scan_chunk(ci, cr, s_v0, d_v0, s_v1, d_v1,
                                      sem_e0, sem_e1),
                lambda cr: scan_chunk(ci, cr, s_v1, d_v1, s_v0, d_v0,
                                      sem_e1, sem_e0),
                carry)

        cnt, f = lax.fori_loop(0, AG_NCHUNK, chunk_body,
                               (jnp.int32(0), jnp.int32(0)))
        cp = lax.bitwise_and(f, 1)

        @pl.when((f > 0) & (cp == 1))
        def _():
            wait_scatter(sbuf0, dbuf0, rows0)

        @pl.when((f > 0) & (cp == 0))
        def _():
            wait_scatter(sbuf1, dbuf1, rows1)

        stage_to(sbuf0, dbuf0)
        pltpu.async_copy(h_hbm.at[sbuf0.at[0]], rows0, sem_g)
        wait_scatter(sbuf0, dbuf0, rows0)
        plsc.subcore_barrier()

        @pl.when(s < 15)
        def _():
            pltpu.sync_copy(acc_sh.at[pl.ds(s * 640, 640)],
                            agg_hbm.at[pl.ds(lo + s * 640, 640)])

        @pl.when(s == 15)
        def _():
            pltpu.sync_copy(acc_sh.at[pl.ds(9600, 400)],
                            agg_hbm.at[pl.ds(lo + 9600, 400)])

        plsc.subcore_barrier()


def sc_agg(h, s_pad, d_pad):
    kern = pl.kernel(
        _agg_body,
        compiler_params=pltpu.CompilerParams(needs_layout_passes=False),
        out_type=jax.ShapeDtypeStruct((NS, LAT), jnp.float32),
        mesh=_sc_mesh(),
        scratch_types=[
            pltpu.VMEM((CH_E,), jnp.int32),
            pltpu.VMEM((CH_E,), jnp.int32),
            pltpu.VMEM((CH_E,), jnp.int32),
            pltpu.VMEM((CH_E,), jnp.int32),
            pltpu.VMEM((1, 128), jnp.int32),
            pltpu.VMEM((1, 128), jnp.int32),
            pltpu.VMEM((1, 128), jnp.int32),
            pltpu.VMEM((1, 128), jnp.int32),
            pltpu.VMEM((1, 128), jnp.int32),
            pltpu.VMEM((1, 128), jnp.int32),
            pltpu.VMEM((AG_C, LAT), jnp.float32),
            pltpu.VMEM((AG_C, LAT), jnp.float32),
            pltpu.VMEM((32, LAT), jnp.float32),
            pltpu.VMEM_SHARED((ZACC_R, LAT), jnp.float32),
            pltpu.SemaphoreType.DMA,
            pltpu.SemaphoreType.DMA,
            pltpu.SemaphoreType.DMA,
        ],
    )
    return kern(h, s_pad, d_pad)


E_PAD = 163840
W_CH = E_PAD // 32
M_CH = E_PAD // 16
M_SUB = 1024


def _w_body(px_h, py_h, pz_h, es_h, ed_h, w_h, px_v, py_v, pz_v,
            es_v, ed_v, w_v, sem):
    c = lax.axis_index("c")
    s = lax.axis_index("s")
    wid = c * 16 + s
    base = wid * W_CH
    pltpu.async_copy(px_h, px_v, sem).wait()
    pltpu.async_copy(py_h, py_v, sem).wait()
    pltpu.async_copy(pz_h, pz_v, sem).wait()
    pltpu.async_copy(es_h.at[pl.ds(base, W_CH)], es_v, sem).wait()
    pltpu.async_copy(ed_h.at[pl.ds(base, W_CH)], ed_v, sem).wait()

    @pl.loop(0, W_CH // 16)
    def _(i):
        esv = es_v[pl.ds(i * 16, 16)]
        edv = ed_v[pl.ds(i * 16, 16)]
        dx = plsc.load_gather(px_v, [esv]) - plsc.load_gather(px_v, [edv])
        dy = plsc.load_gather(py_v, [esv]) - plsc.load_gather(py_v, [edv])
        dz = plsc.load_gather(pz_v, [esv]) - plsc.load_gather(pz_v, [edv])
        w_v[pl.ds(i * 16, 16)] = jnp.exp(-(dx * dx + dy * dy + dz * dz))

    pltpu.sync_copy(w_v, w_h.at[pl.ds(base, W_CH)])


def sc_w(px, py, pz, es_w, ed_w):
    kern = pl.kernel(
        _w_body,
        compiler_params=pltpu.CompilerParams(needs_layout_passes=False),
        out_type=jax.ShapeDtypeStruct((E_PAD,), jnp.float32),
        mesh=_sc_mesh(),
        scratch_types=[
            pltpu.VMEM((N,), jnp.float32),
            pltpu.VMEM((N,), jnp.float32),
            pltpu.VMEM((N,), jnp.float32),
            pltpu.VMEM((W_CH,), jnp.int32),
            pltpu.VMEM((W_CH,), jnp.int32),
            pltpu.VMEM((W_CH,), jnp.float32),
            pltpu.SemaphoreType.DMA,
        ],
    )
    return kern(px, py, pz, es_w, ed_w)


def _m_body(xn_h, es_h, ed_h, w_h, m_h, es_c, ed_c, w_c, rows_v, zb_v,
            acc_sh, sem):
    c = lax.axis_index("c")
    s = lax.axis_index("s")
    zeros16 = jnp.zeros((16,), jnp.float32)

    @pl.loop(0, 32)
    def _(r):
        @pl.loop(0, 8)
        def _(q):
            zb_v[r, pl.ds(q * 16, 16)] = zeros16

    @pl.loop(0, ZACC_R // 16 // 32)
    def _(j):
        pltpu.sync_copy(zb_v, acc_sh.at[pl.ds(s * (ZACC_R // 16) + j * 32, 32)])
    plsc.subcore_barrier()

    @pl.loop(0, M_CH // M_SUB)
    def _(q):
        row0 = s * (M_CH // 128) + q * (M_SUB // 128)
        pltpu.async_copy(es_h.at[pl.ds(row0, M_SUB // 128)], es_c, sem).wait()
        pltpu.async_copy(ed_h.at[pl.ds(row0, M_SUB // 128)], ed_c, sem).wait()
        pltpu.async_copy(w_h.at[pl.ds(s * M_CH + q * M_SUB, M_SUB)],
                         w_c, sem).wait()

        @pl.loop(0, M_SUB // 128)
        def _(j):
            pltpu.async_copy(xn_h.at[c].at[es_c.at[j]], rows_v, sem).wait()

            @pl.loop(0, 128)
            def _(r):
                wsp = plsc.load_gather(w_c, [jnp.full((16,), j * 128 + r,
                                                      jnp.int32)])

                @pl.loop(0, 8)
                def _(k):
                    rows_v[r, pl.ds(k * 16, 16)] = (
                        rows_v[r, pl.ds(k * 16, 16)] * wsp)

            pltpu.sync_copy(rows_v, acc_sh.at[ed_c.at[j]], add=True)

    plsc.subcore_barrier()

    @pl.when(s < 15)
    def _():
        pltpu.sync_copy(acc_sh.at[pl.ds(s * 640, 640)],
                        m_h.at[c, pl.ds(s * 640, 640)])

    @pl.when(s == 15)
    def _():
        pltpu.sync_copy(acc_sh.at[pl.ds(9600, 640)],
                        m_h.at[c, pl.ds(9600, 640)])

    plsc.subcore_barrier()


def sc_m(xn2, es2d, ed2d, w):
    kern = pl.kernel(
        _m_body,
        compiler_params=pltpu.CompilerParams(needs_layout_passes=False),
        out_type=jax.ShapeDtypeStruct((2, ZACC_R, 128), jnp.float32),
        mesh=_sc_mesh(),
        scratch_types=[
            pltpu.VMEM((M_SUB // 128, 128), jnp.int32),
            pltpu.VMEM((M_SUB // 128, 128), jnp.int32),
            pltpu.VMEM((M_SUB,), jnp.float32),
            pltpu.VMEM((128, 128), jnp.float32),
            pltpu.VMEM((32, 128), jnp.float32),
            pltpu.VMEM_SHARED((ZACC_R, 128), jnp.float32),
            pltpu.SemaphoreType.DMA,
        ],
    )
    return kern(xn2, es2d, ed2d, w)


def kernel(x, adj_t, pos, batch, sub_x, sub_adj_t, sub_batch, edge_index,
           batch_lengths, edge_batch, We1, be1, We2, be2, gamma, beta,
           Wg1, bg1, Wg2, bg2):
    h = mm_bias(sub_x, We1, be1, relu=True)
    s_pad = jnp.pad(sub_adj_t[0].astype(jnp.int32), (0, ES_PAD - ES))
    d_pad = jnp.pad(sub_adj_t[1].astype(jnp.int32), (0, ES_PAD - ES),
                    constant_values=2 ** 30)
    agg = sc_agg(h, s_pad, d_pad)
    h2 = mm_bias(agg, We2, be2, relu=True, out_rows=NS_PAD)
    sb_pad = jnp.pad(sub_batch.astype(jnp.int32), (0, NS_PAD - NS),
                     constant_values=ZPAD_IDX)
    zcnt = sc_zs(h2, sb_pad)
    cnts = sc_cnt(sb_pad)
    xz, sums, sq = bn_stats(x, zcnt, cnts)
    xn2 = bn_apply(xz, sums, sq, gamma, beta)
    es = edge_index[0].astype(jnp.int32)
    ed = edge_index[1].astype(jnp.int32)
    es_w = jnp.pad(es, (0, E_PAD - E))
    ed_w = jnp.pad(ed, (0, E_PAD - E))
    w = sc_w(pos[:, 0], pos[:, 1], pos[:, 2], es_w, ed_w)
    ed_m = jnp.pad(ed, (0, E_PAD - E), constant_values=ZPAD_IDX)
    m2 = sc_m(xn2, es_w.reshape(E_PAD // 128, 128),
              ed_m.reshape(E_PAD // 128, 128), w)
    hg = mm_split(m2, Wg1, bg1)
    return pool_out(hg, batch, Wg2, bg2)


# speedup vs baseline: 3.8424x; 1.0937x over previous
import functools

import jax
import jax.numpy as jnp
from jax import lax
from jax.experimental import pallas as pl
from jax.experimental.pallas import tpu as pltpu
from jax.experimental.pallas import tpu_sc as plsc

N, D, LAT, NS, ES, E, G = 10000, 128, 128, 100000, 400000, 160000, 64
F = D + LAT


def _mm_split_body(a_ref, w_ref, b_ref, o_ref):
    acc = jnp.dot(a_ref[0], w_ref[:128, :], preferred_element_type=jnp.float32)
    acc += jnp.dot(a_ref[1], w_ref[128:, :], preferred_element_type=jnp.float32)
    o_ref[...] = jnp.maximum(acc + b_ref[...], 0.0)


def mm_split(m2, w, b, block_m=2000):
    n = w.shape[1]
    return pl.pallas_call(
        _mm_split_body,
        grid=(N // block_m,),
        in_specs=[
            pl.BlockSpec((2, block_m, 128), lambda i: (0, i, 0)),
            pl.BlockSpec((F, n), lambda i: (0, 0)),
            pl.BlockSpec((1, n), lambda i: (0, 0)),
        ],
        out_specs=pl.BlockSpec((block_m, n), lambda i: (i, 0)),
        out_shape=jax.ShapeDtypeStruct((N, n), jnp.float32),
    )(m2, w, b.reshape(1, n))


def _mm_bias_body(a_ref, w_ref, b_ref, o_ref, *, relu):
    acc = jnp.dot(a_ref[...], w_ref[...], preferred_element_type=jnp.float32)
    acc = acc + b_ref[...]
    if relu:
        acc = jnp.maximum(acc, 0.0)
    o_ref[...] = acc


def mm_bias(a, w, b, relu, block_m=2000, out_rows=None):
    m, k = a.shape
    n = w.shape[1]
    assert m % block_m == 0
    return pl.pallas_call(
        functools.partial(_mm_bias_body, relu=relu),
        grid=(m // block_m,),
        in_specs=[
            pl.BlockSpec((block_m, k), lambda i: (i, 0)),
            pl.BlockSpec((k, n), lambda i: (0, 0)),
            pl.BlockSpec((1, n), lambda i: (0, 0)),
        ],
        out_specs=pl.BlockSpec((block_m, n), lambda i: (i, 0)),
        out_shape=jax.ShapeDtypeStruct((out_rows or m, n), jnp.float32),
    )(a, w, b.reshape(1, n))


def _bn_stats_body(x_ref, zc_ref, ct_ref, xz_ref, sums_ref, sq_ref,
                   acc_s, acc_q, *, nsteps):
    i = pl.program_id(0)

    @pl.when(i == 0)
    def _():
        acc_s[...] = jnp.zeros_like(acc_s)
        acc_q[...] = jnp.zeros_like(acc_q)

    zc = zc_ref[0] + zc_ref[1]
    ct = ct_ref[0, :, 0:1] + ct_ref[1, :, 0:1]
    z = zc / jnp.maximum(ct, 1.0)
    xz = jnp.concatenate((x_ref[...], z), axis=1)
    xz_ref[...] = xz
    acc_s[...] += jnp.sum(xz, axis=0, keepdims=True)
    acc_q[...] += jnp.sum(xz * xz, axis=0, keepdims=True)

    @pl.when(i == nsteps - 1)
    def _():
        sums_ref[...] = acc_s[...]
        sq_ref[...] = acc_q[...]


def bn_stats(x, zcnt, cnts, block_m=2000):
    nsteps = N // block_m
    return pl.pallas_call(
        functools.partial(_bn_stats_body, nsteps=nsteps),
        grid=(nsteps,),
        in_specs=[
            pl.BlockSpec((block_m, D), lambda i: (i, 0)),
            pl.BlockSpec((2, block_m, LAT), lambda i: (0, i, 0)),
            pl.BlockSpec((2, block_m, LAT), lambda i: (0, i, 0)),
        ],
        out_specs=[
            pl.BlockSpec((block_m, F), lambda i: (i, 0)),
            pl.BlockSpec((1, F), lambda i: (0, 0)),
            pl.BlockSpec((1, F), lambda i: (0, 0)),
        ],
        out_shape=[
            jax.ShapeDtypeStruct((N, F), jnp.float32),
            jax.ShapeDtypeStruct((1, F), jnp.float32),
            jax.ShapeDtypeStruct((1, F), jnp.float32),
        ],
        scratch_shapes=[
            pltpu.VMEM((1, F), jnp.float32),
            pltpu.VMEM((1, F), jnp.float32),
        ],
    )(x, zcnt, cnts)


def _bn_apply_body(xz_ref, s_ref, q_ref, g_ref, b_ref, o_ref):
    mu = s_ref[...] / N
    var = q_ref[...] / N - mu * mu
    rstd = jax.lax.rsqrt(var + 1e-5)
    xn = (xz_ref[...] - mu) * rstd * g_ref[...] + b_ref[...]
    o_ref[0] = xn[:, :128]
    o_ref[1] = xn[:, 128:]


def bn_apply(xz, sums, sq, gamma, beta, block_m=2000):
    return pl.pallas_call(
        _bn_apply_body,
        grid=(N // block_m,),
        in_specs=[
            pl.BlockSpec((block_m, F), lambda i: (i, 0)),
            pl.BlockSpec((1, F), lambda i: (0, 0)),
            pl.BlockSpec((1, F), lambda i: (0, 0)),
            pl.BlockSpec((1, F), lambda i: (0, 0)),
            pl.BlockSpec((1, F), lambda i: (0, 0)),
        ],
        out_specs=pl.BlockSpec((2, block_m, 128), lambda i: (0, i, 0)),
        out_shape=jax.ShapeDtypeStruct((2, N, 128), jnp.float32),
    )(xz, sums, sq, gamma.reshape(1, F), beta.reshape(1, F))


def _pool_out_body(hg_ref, batch_ref, w_ref, b_ref, o_ref, acc, cnt,
                   *, block_m, nsteps):
    i = pl.program_id(0)

    @pl.when(i == 0)
    def _():
        acc[...] = jnp.zeros_like(acc)
        cnt[...] = jnp.zeros_like(cnt)

    ids = batch_ref[0, 0, :]
    gids = jax.lax.broadcasted_iota(jnp.int32, (G, block_m), 0)
    onehot = (ids[None, :] == gids).astype(jnp.float32)
    acc[...] += jnp.dot(onehot, hg_ref[...],
                        preferred_element_type=jnp.float32)
    cnt[...] += jnp.sum(onehot, axis=1, keepdims=True)

    @pl.when(i == nsteps - 1)
    def _():
        pooled = acc[...] / jnp.maximum(cnt[...], 1.0)
        o_ref[...] = jnp.dot(pooled, w_ref[...],
                             preferred_element_type=jnp.float32) + b_ref[...]


def pool_out(hg, batch, w, b, block_m=2000):
    nsteps = N // block_m
    return pl.pallas_call(
        functools.partial(_pool_out_body, block_m=block_m, nsteps=nsteps),
        grid=(nsteps,),
        in_specs=[
            pl.BlockSpec((block_m, F), lambda i: (i, 0)),
            pl.BlockSpec((1, 1, block_m), lambda i: (i, 0, 0)),
            pl.BlockSpec((F, 128), lambda i: (0, 0)),
            pl.BlockSpec((1, 128), lambda i: (0, 0)),
        ],
        out_specs=pl.BlockSpec((G, 128), lambda i: (0, 0)),
        out_shape=jax.ShapeDtypeStruct((G, 128), jnp.float32),
        scratch_shapes=[
            pltpu.VMEM((G, F), jnp.float32),
            pltpu.VMEM((G, 1), jnp.float32),
        ],
    )(hg, batch.astype(jnp.int32).reshape(nsteps, 1, block_m), w,
      b.reshape(1, 128))


NS_PAD = 102400
ZS_CH = NS_PAD // 32
ZS_NCHUNK = ZS_CH // 128
ZACC_R = 10240
ZPAD_IDX = 10016

_MESH_CACHE = []


def _sc_mesh():
    if not _MESH_CACHE:
        _MESH_CACHE.append(plsc.VectorSubcoreMesh(core_axis_name="c",
                                                  subcore_axis_name="s"))
    return _MESH_CACHE[0]


def _zs_body(h2_hbm, sb_hbm, out_hbm, idx_v, rows_v, zb_v, acc_sh, sem):
    c = lax.axis_index("c")
    s = lax.axis_index("s")
    wid = c * 16 + s
    base = wid * ZS_CH

    zb_v[...] = jnp.zeros_like(zb_v)
    @pl.loop(0, ZACC_R // 16 // 64)
    def _(j):
        pltpu.sync_copy(zb_v, acc_sh.at[pl.ds(s * (ZACC_R // 16) + j * 64, 64)])
    pltpu.async_copy(sb_hbm.at[wid], idx_v, sem).wait()
    plsc.subcore_barrier()

    @pl.loop(0, ZS_NCHUNK)
    def _(j):
        pltpu.async_copy(h2_hbm.at[pl.ds(base + j * 128, 128)],
                         rows_v, sem).wait()
        pltpu.sync_copy(rows_v, acc_sh.at[idx_v.at[j]], add=True)

    plsc.subcore_barrier()
    pltpu.sync_copy(acc_sh.at[pl.ds(s * 640, 640)],
                    out_hbm.at[c, pl.ds(s * 640, 640)])


def sc_zs(h2, sub_batch_pad):
    kern = pl.kernel(
        _zs_body,
        out_type=jax.ShapeDtypeStruct((2, ZACC_R, LAT), jnp.float32),
        mesh=_sc_mesh(),
        scratch_types=[
            pltpu.VMEM((ZS_NCHUNK, 128), jnp.int32),
            pltpu.VMEM((128, LAT), jnp.float32),
            pltpu.VMEM((64, LAT), jnp.float32),
            pltpu.VMEM_SHARED((ZACC_R, LAT), jnp.float32),
            pltpu.SemaphoreType.DMA,
        ],
    )
    return kern(h2, sub_batch_pad.reshape(32, ZS_NCHUNK, 128))


def _cnt_body(sb_hbm, out_hbm, idx_v, ones_v, zb_v, acc_sh, sem):
    c = lax.axis_index("c")
    s = lax.axis_index("s")
    wid = c * 16 + s

    zb_v[...] = jnp.zeros_like(zb_v)
    @pl.loop(0, ZACC_R // 16 // 64)
    def _(j):
        pltpu.sync_copy(zb_v, acc_sh.at[pl.ds(s * (ZACC_R // 16) + j * 64, 64)])
    ones_v[...] = jnp.ones_like(ones_v)
    pltpu.async_copy(sb_hbm.at[wid], idx_v, sem).wait()
    plsc.subcore_barrier()

    @pl.loop(0, ZS_NCHUNK)
    def _(j):
        pltpu.sync_copy(ones_v, acc_sh.at[idx_v.at[j]], add=True)

    plsc.subcore_barrier()
    pltpu.sync_copy(acc_sh.at[pl.ds(s * 640, 640)],
                    out_hbm.at[c, pl.ds(s * 640, 640)])


def sc_cnt(sub_batch_pad):
    kern = pl.kernel(
        _cnt_body,
        out_type=jax.ShapeDtypeStruct((2, ZACC_R, LAT), jnp.float32),
        mesh=_sc_mesh(),
        scratch_types=[
            pltpu.VMEM((ZS_NCHUNK, 128), jnp.int32),
            pltpu.VMEM((128, LAT), jnp.float32),
            pltpu.VMEM((64, LAT), jnp.float32),
            pltpu.VMEM_SHARED((ZACC_R, LAT), jnp.float32),
            pltpu.SemaphoreType.DMA,
        ],
    )
    return kern(sub_batch_pad.reshape(32, ZS_NCHUNK, 128))


ES_PAD = 425984
AG_CH = ES_PAD // 16
CH_E = 2048
AG_NCHUNK = AG_CH // CH_E
AG_R = 10000
AG_PASSES = 5
AG_C = 128
AG_THRESH = AG_C - 16


def _agg_body(h_hbm, s_hbm, d_hbm, agg_hbm, s_v0, d_v0, s_v1, d_v1,
              sbufc, dbufc, sbuf0, dbuf0, sbuf1, dbuf1, rows0, rows1,
              zb_v, acc_sh, sem_e0, sem_e1, sem_g):
    c = lax.axis_index("c")
    s = lax.axis_index("s")
    wid = c * 16 + s
    lane = lax.iota(jnp.int32, 16)
    dummy_src = wid * 16 + lane
    dummy_dst = 10000 + s * 14 + lane
    zeros16 = jnp.zeros((16,), jnp.float32)
    ebase = s * AG_CH

    def refill_c():
        @pl.loop(0, 8)
        def _(q):
            sbufc[0, pl.ds(q * 16, 16)] = dummy_src
            dbufc[0, pl.ds(q * 16, 16)] = dummy_dst

    def stage_to(sb, db):
        @pl.loop(0, 8)
        def _(q):
            sb[0, pl.ds(q * 16, 16)] = sbufc[0, pl.ds(q * 16, 16)]
            db[0, pl.ds(q * 16, 16)] = dbufc[0, pl.ds(q * 16, 16)]
        refill_c()

    def wait_scatter(sb, db, rows):
        pltpu.make_async_copy(h_hbm.at[sb.at[0]], rows, sem_g).wait()
        pltpu.sync_copy(rows, acc_sh.at[db.at[0]], add=True)

    @pl.loop(0, 32)
    def _(r):
        @pl.loop(0, 8)
        def _(q):
            zb_v[r, pl.ds(q * 16, 16)] = zeros16
    refill_c()

    for p in range(AG_PASSES):
        lo = c * (AG_PASSES * AG_R) + p * AG_R

        @pl.loop(0, ZACC_R // 16 // 32)
        def _(j):
            pltpu.sync_copy(zb_v,
                            acc_sh.at[pl.ds(s * (ZACC_R // 16) + j * 32, 32)])
        plsc.subcore_barrier()

        pltpu.async_copy(s_hbm.at[pl.ds(ebase, CH_E)], s_v0, sem_e0)
        pltpu.async_copy(d_hbm.at[pl.ds(ebase, CH_E)], d_v0, sem_e0)

        def scan_chunk(ci, carry, sv_ref, dv_ref, nsv_ref, ndv_ref,
                       sem_cur, sem_nxt):
            pltpu.make_async_copy(s_hbm.at[pl.ds(ebase + ci * CH_E, CH_E)],
                                  sv_ref, sem_cur).wait()
            pltpu.make_async_copy(d_hbm.at[pl.ds(ebase + ci * CH_E, CH_E)],
                                  dv_ref, sem_cur).wait()

            @pl.when(ci < AG_NCHUNK - 1)
            def _():
                pltpu.async_copy(
                    s_hbm.at[pl.ds(ebase + (ci + 1) * CH_E, CH_E)],
                    nsv_ref, sem_nxt)
                pltpu.async_copy(
                    d_hbm.at[pl.ds(ebase + (ci + 1) * CH_E, CH_E)],
                    ndv_ref, sem_nxt)

            def body(i, cf):
                cnt, f = cf
                dv = dv_ref[pl.ds(i * 16, 16)]
                sv = sv_ref[pl.ds(i * 16, 16)]
                ldv = dv - lo
                m = (ldv >= 0) & (ldv < AG_R)
                pos = cnt + plsc.cumsum(m.astype(jnp.int32)) - 1
                plsc.store_scatter(sbufc, [pos - pos, pos], sv, mask=m)
                plsc.store_scatter(dbufc, [pos - pos, pos], ldv, mask=m)
                newcnt = cnt + jnp.max(plsc.all_reduce_population_count(m))
                do_flush = newcnt >= AG_THRESH
                cp = lax.bitwise_and(f, 1)

                @pl.when(do_flush & (cp == 0))
                def _():
                    @pl.when(f > 0)
                    def _():
                        wait_scatter(sbuf1, dbuf1, rows1)
                    stage_to(sbuf0, dbuf0)
                    pltpu.async_copy(h_hbm.at[sbuf0.at[0]], rows0, sem_g)

                @pl.when(do_flush & (cp == 1))
                def _():
                    wait_scatter(sbuf0, dbuf0, rows0)
                    stage_to(sbuf1, dbuf1)
                    pltpu.async_copy(h_hbm.at[sbuf1.at[0]], rows1, sem_g)

                return (jnp.where(do_flush, 0, newcnt),
                        jnp.where(do_flush, f + 1, f))

            return lax.fori_loop(0, CH_E // 16, body, carry)

        def chunk_body(ci, carry):
            return lax.cond(
                lax.bitwise_and(ci, 1) == 0,
                lambda cr: scan_chunk(ci, cr, s_v0, d_v0, s_v1, d_v1,
                                      sem_e0, sem_e1),
                lambda cr: scan_chunk(ci, cr, s_v1, d_v1, s_v0, d_v0,
                                      sem_e1, sem_e0),
                carry)

        cnt, f = lax.fori_loop(0, AG_NCHUNK, chunk_body,
                               (jnp.int32(0), jnp.int32(0)))
        cp = lax.bitwise_and(f, 1)

        @pl.when((f > 0) & (cp == 1))
        def _():
            wait_scatter(sbuf0, dbuf0, rows0)

        @pl.when((f > 0) & (cp == 0))
        def _():
            wait_scatter(sbuf1, dbuf1, rows1)

        stage_to(sbuf0, dbuf0)
        pltpu.async_copy(h_hbm.at[sbuf0.at[0]], rows0, sem_g)
        wait_scatter(sbuf0, dbuf0, rows0)
        plsc.subcore_barrier()

        @pl.when(s < 15)
        def _():
            pltpu.sync_copy(acc_sh.at[pl.ds(s * 640, 640)],
                            agg_hbm.at[pl.ds(lo + s * 640, 640)])

        @pl.when(s == 15)
        def _():
            pltpu.sync_copy(acc_sh.at[pl.ds(9600, 400)],
                            agg_hbm.at[pl.ds(lo + 9600, 400)])

        plsc.subcore_barrier()


def sc_agg(h, s_pad, d_pad):
    kern = pl.kernel(
        _agg_body,
        compiler_params=pltpu.CompilerParams(needs_layout_passes=False),
        out_type=jax.ShapeDtypeStruct((NS, LAT), jnp.float32),
        mesh=_sc_mesh(),
        scratch_types=[
            pltpu.VMEM((CH_E,), jnp.int32),
            pltpu.VMEM((CH_E,), jnp.int32),
            pltpu.VMEM((CH_E,), jnp.int32),
            pltpu.VMEM((CH_E,), jnp.int32),
            pltpu.VMEM((1, 128), jnp.int32),
            pltpu.VMEM((1, 128), jnp.int32),
            pltpu.VMEM((1, 128), jnp.int32),
            pltpu.VMEM((1, 128), jnp.int32),
            pltpu.VMEM((1, 128), jnp.int32),
            pltpu.VMEM((1, 128), jnp.int32),
            pltpu.VMEM((AG_C, LAT), jnp.float32),
            pltpu.VMEM((AG_C, LAT), jnp.float32),
            pltpu.VMEM((32, LAT), jnp.float32),
            pltpu.VMEM_SHARED((ZACC_R, LAT), jnp.float32),
            pltpu.SemaphoreType.DMA,
            pltpu.SemaphoreType.DMA,
            pltpu.SemaphoreType.DMA,
        ],
    )
    return kern(h, s_pad, d_pad)


E_PAD = 163840
W_CH = E_PAD // 32
M_CH = E_PAD // 16
M_SUB = 1024


def _w_body(px_h, py_h, pz_h, es_h, ed_h, w_h, px_v, py_v, pz_v,
            es_v, ed_v, w_v, sem):
    c = lax.axis_index("c")
    s = lax.axis_index("s")
    wid = c * 16 + s
    base = wid * W_CH
    pltpu.async_copy(px_h, px_v, sem).wait()
    pltpu.async_copy(py_h, py_v, sem).wait()
    pltpu.async_copy(pz_h, pz_v, sem).wait()
    pltpu.async_copy(es_h.at[pl.ds(base, W_CH)], es_v, sem).wait()
    pltpu.async_copy(ed_h.at[pl.ds(base, W_CH)], ed_v, sem).wait()

    @pl.loop(0, W_CH // 16)
    def _(i):
        esv = es_v[pl.ds(i * 16, 16)]
        edv = ed_v[pl.ds(i * 16, 16)]
        dx = plsc.load_gather(px_v, [esv]) - plsc.load_gather(px_v, [edv])
        dy = plsc.load_gather(py_v, [esv]) - plsc.load_gather(py_v, [edv])
        dz = plsc.load_gather(pz_v, [esv]) - plsc.load_gather(pz_v, [edv])
        w_v[pl.ds(i * 16, 16)] = jnp.exp(-(dx * dx + dy * dy + dz * dz))

    pltpu.sync_copy(w_v, w_h.at[pl.ds(base, W_CH)])


def sc_w(px, py, pz, es_w, ed_w):
    kern = pl.kernel(
        _w_body,
        compiler_params=pltpu.CompilerParams(needs_layout_passes=False),
        out_type=jax.ShapeDtypeStruct((E_PAD,), jnp.float32),
        mesh=_sc_mesh(),
        scratch_types=[
            pltpu.VMEM((N,), jnp.float32),
            pltpu.VMEM((N,), jnp.float32),
            pltpu.VMEM((N,), jnp.float32),
            pltpu.VMEM((W_CH,), jnp.int32),
            pltpu.VMEM((W_CH,), jnp.int32),
            pltpu.VMEM((W_CH,), jnp.float32),
            pltpu.SemaphoreType.DMA,
        ],
    )
    return kern(px, py, pz, es_w, ed_w)


def _m_body(xn_h, es_h, ed_h, w_h, m_h, es_c, ed_c, w_c, rows0, rows1,
            zb_v, acc_sh, sem_e, sem_g, sem_s):
    c = lax.axis_index("c")
    s = lax.axis_index("s")
    zeros16 = jnp.zeros((16,), jnp.float32)
    rows = (rows0, rows1)

    @pl.loop(0, 32)
    def _(r):
        @pl.loop(0, 8)
        def _(q):
            zb_v[r, pl.ds(q * 16, 16)] = zeros16

    @pl.loop(0, ZACC_R // 16 // 32)
    def _(j):
        pltpu.sync_copy(zb_v, acc_sh.at[pl.ds(s * (ZACC_R // 16) + j * 32, 32)])
    plsc.subcore_barrier()

    def scale(rbuf, base):
        @pl.loop(0, 128)
        def _(r):
            wsp = plsc.load_gather(w_c, [jnp.full((16,), base + r, jnp.int32)])

            @pl.loop(0, 8)
            def _(k):
                rbuf[r, pl.ds(k * 16, 16)] = rbuf[r, pl.ds(k * 16, 16)] * wsp

    @pl.loop(0, M_CH // M_SUB)
    def _(q):
        row0 = s * (M_CH // 128) + q * (M_SUB // 128)
        pltpu.async_copy(es_h.at[pl.ds(row0, M_SUB // 128)], es_c, sem_e)
        pltpu.async_copy(ed_h.at[pl.ds(row0, M_SUB // 128)], ed_c, sem_e)
        pltpu.async_copy(w_h.at[pl.ds(s * M_CH + q * M_SUB, M_SUB)],
                         w_c, sem_e)
        pltpu.make_async_copy(es_h.at[pl.ds(row0, M_SUB // 128)],
                              es_c, sem_e).wait()
        pltpu.make_async_copy(ed_h.at[pl.ds(row0, M_SUB // 128)],
                              ed_c, sem_e).wait()
        pltpu.make_async_copy(w_h.at[pl.ds(s * M_CH + q * M_SUB, M_SUB)],
                              w_c, sem_e).wait()

        pltpu.async_copy(xn_h.at[c].at[es_c.at[0]], rows0, sem_g)
        for j in range(M_SUB // 128):
            b = j % 2
            pltpu.make_async_copy(xn_h.at[c].at[es_c.at[j]],
                                  rows[b], sem_g).wait()
            if j < M_SUB // 128 - 1:
                if j >= 1:
                    pltpu.make_async_copy(rows[1 - b],
                                          acc_sh.at[ed_c.at[j - 1]],
                                          sem_s).wait()
                pltpu.async_copy(xn_h.at[c].at[es_c.at[j + 1]],
                                 rows[1 - b], sem_g)
            scale(rows[b], j * 128)
            pltpu.async_copy(rows[b], acc_sh.at[ed_c.at[j]], sem_s, add=True)
        pltpu.make_async_copy(rows[0], acc_sh.at[ed_c.at[6]], sem_s).wait()
        pltpu.make_async_copy(rows[1], acc_sh.at[ed_c.at[7]], sem_s).wait()

    plsc.subcore_barrier()

    @pl.when(s < 15)
    def _():
        pltpu.sync_copy(acc_sh.at[pl.ds(s * 640, 640)],
                        m_h.at[c, pl.ds(s * 640, 640)])

    @pl.when(s == 15)
    def _():
        pltpu.sync_copy(acc_sh.at[pl.ds(9600, 640)],
                        m_h.at[c, pl.ds(9600, 640)])

    plsc.subcore_barrier()


def sc_m(xn2, es2d, ed2d, w):
    kern = pl.kernel(
        _m_body,
        compiler_params=pltpu.CompilerParams(needs_layout_passes=False),
        out_type=jax.ShapeDtypeStruct((2, ZACC_R, 128), jnp.float32),
        mesh=_sc_mesh(),
        scratch_types=[
            pltpu.VMEM((M_SUB // 128, 128), jnp.int32),
            pltpu.VMEM((M_SUB // 128, 128), jnp.int32),
            pltpu.VMEM((M_SUB,), jnp.float32),
            pltpu.VMEM((128, 128), jnp.float32),
            pltpu.VMEM((128, 128), jnp.float32),
            pltpu.VMEM((32, 128), jnp.float32),
            pltpu.VMEM_SHARED((ZACC_R, 128), jnp.float32),
            pltpu.SemaphoreType.DMA,
            pltpu.SemaphoreType.DMA,
            pltpu.SemaphoreType.DMA,
        ],
    )
    return kern(xn2, es2d, ed2d, w)


def kernel(x, adj_t, pos, batch, sub_x, sub_adj_t, sub_batch, edge_index,
           batch_lengths, edge_batch, We1, be1, We2, be2, gamma, beta,
           Wg1, bg1, Wg2, bg2):
    h = mm_bias(sub_x, We1, be1, relu=True)
    s_pad = jnp.pad(sub_adj_t[0].astype(jnp.int32), (0, ES_PAD - ES))
    d_pad = jnp.pad(sub_adj_t[1].astype(jnp.int32), (0, ES_PAD - ES),
                    constant_values=2 ** 30)
    agg = sc_agg(h, s_pad, d_pad)
    h2 = mm_bias(agg, We2, be2, relu=True, out_rows=NS_PAD)
    sb_pad = jnp.pad(sub_batch.astype(jnp.int32), (0, NS_PAD - NS),
                     constant_values=ZPAD_IDX)
    zcnt = sc_zs(h2, sb_pad)
    cnts = sc_cnt(sb_pad)
    xz, sums, sq = bn_stats(x, zcnt, cnts)
    xn2 = bn_apply(xz, sums, sq, gamma, beta)
    es = edge_index[0].astype(jnp.int32)
    ed = edge_index[1].astype(jnp.int32)
    es_w = jnp.pad(es, (0, E_PAD - E))
    ed_w = jnp.pad(ed, (0, E_PAD - E))
    w = sc_w(pos[:, 0], pos[:, 1], pos[:, 2], es_w, ed_w)
    ed_m = jnp.pad(ed, (0, E_PAD - E), constant_values=ZPAD_IDX)
    m2 = sc_m(xn2, es_w.reshape(E_PAD // 128, 128),
              ed_m.reshape(E_PAD // 128, 128), w)
    hg = mm_split(m2, Wg1, bg1)
    return pool_out(hg, batch, Wg2, bg2)


# speedup vs baseline: 3.8912x; 1.0127x over previous
import functools

import jax
import jax.numpy as jnp
from jax import lax
from jax.experimental import pallas as pl
from jax.experimental.pallas import tpu as pltpu
from jax.experimental.pallas import tpu_sc as plsc

N, D, LAT, NS, ES, E, G = 10000, 128, 128, 100000, 400000, 160000, 64
F = D + LAT


def _mm_split_body(a_ref, w_ref, b_ref, o_ref):
    acc = jnp.dot(a_ref[0], w_ref[:128, :], preferred_element_type=jnp.float32)
    acc += jnp.dot(a_ref[1], w_ref[128:, :], preferred_element_type=jnp.float32)
    o_ref[...] = jnp.maximum(acc + b_ref[...], 0.0)


def mm_split(m2, w, b, block_m=2000):
    n = w.shape[1]
    return pl.pallas_call(
        _mm_split_body,
        grid=(N // block_m,),
        in_specs=[
            pl.BlockSpec((2, block_m, 128), lambda i: (0, i, 0)),
            pl.BlockSpec((F, n), lambda i: (0, 0)),
            pl.BlockSpec((1, n), lambda i: (0, 0)),
        ],
        out_specs=pl.BlockSpec((block_m, n), lambda i: (i, 0)),
        out_shape=jax.ShapeDtypeStruct((N, n), jnp.float32),
    )(m2, w, b.reshape(1, n))


def _mm_bias_body(a_ref, w_ref, b_ref, o_ref, *, relu):
    acc = jnp.dot(a_ref[...], w_ref[...], preferred_element_type=jnp.float32)
    acc = acc + b_ref[...]
    if relu:
        acc = jnp.maximum(acc, 0.0)
    o_ref[...] = acc


def mm_bias(a, w, b, relu, block_m=2000, out_rows=None):
    m, k = a.shape
    n = w.shape[1]
    assert m % block_m == 0
    return pl.pallas_call(
        functools.partial(_mm_bias_body, relu=relu),
        grid=(m // block_m,),
        in_specs=[
            pl.BlockSpec((block_m, k), lambda i: (i, 0)),
            pl.BlockSpec((k, n), lambda i: (0, 0)),
            pl.BlockSpec((1, n), lambda i: (0, 0)),
        ],
        out_specs=pl.BlockSpec((block_m, n), lambda i: (i, 0)),
        out_shape=jax.ShapeDtypeStruct((out_rows or m, n), jnp.float32),
    )(a, w, b.reshape(1, n))


def _bn_stats_body(x_ref, zc_ref, ct_ref, xz_ref, sums_ref, sq_ref,
                   acc_s, acc_q, *, nsteps):
    i = pl.program_id(0)

    @pl.when(i == 0)
    def _():
        acc_s[...] = jnp.zeros_like(acc_s)
        acc_q[...] = jnp.zeros_like(acc_q)

    zc = zc_ref[0] + zc_ref[1]
    ct = ct_ref[0, :, 0:1] + ct_ref[1, :, 0:1]
    z = zc / jnp.maximum(ct, 1.0)
    xz = jnp.concatenate((x_ref[...], z), axis=1)
    xz_ref[...] = xz
    acc_s[...] += jnp.sum(xz, axis=0, keepdims=True)
    acc_q[...] += jnp.sum(xz * xz, axis=0, keepdims=True)

    @pl.when(i == nsteps - 1)
    def _():
        sums_ref[...] = acc_s[...]
        sq_ref[...] = acc_q[...]


def bn_stats(x, zcnt, cnts, block_m=2000):
    nsteps = N // block_m
    return pl.pallas_call(
        functools.partial(_bn_stats_body, nsteps=nsteps),
        grid=(nsteps,),
        in_specs=[
            pl.BlockSpec((block_m, D), lambda i: (i, 0)),
            pl.BlockSpec((2, block_m, LAT), lambda i: (0, i, 0)),
            pl.BlockSpec((2, block_m, LAT), lambda i: (0, i, 0)),
        ],
        out_specs=[
            pl.BlockSpec((block_m, F), lambda i: (i, 0)),
            pl.BlockSpec((1, F), lambda i: (0, 0)),
            pl.BlockSpec((1, F), lambda i: (0, 0)),
        ],
        out_shape=[
            jax.ShapeDtypeStruct((N, F), jnp.float32),
            jax.ShapeDtypeStruct((1, F), jnp.float32),
            jax.ShapeDtypeStruct((1, F), jnp.float32),
        ],
        scratch_shapes=[
            pltpu.VMEM((1, F), jnp.float32),
            pltpu.VMEM((1, F), jnp.float32),
        ],
    )(x, zcnt, cnts)


def _bn_apply_body(xz_ref, s_ref, q_ref, g_ref, b_ref, o_ref):
    mu = s_ref[...] / N
    var = q_ref[...] / N - mu * mu
    rstd = jax.lax.rsqrt(var + 1e-5)
    xn = (xz_ref[...] - mu) * rstd * g_ref[...] + b_ref[...]
    o_ref[0] = xn[:, :128]
    o_ref[1] = xn[:, 128:]


def bn_apply(xz, sums, sq, gamma, beta, block_m=2000):
    return pl.pallas_call(
        _bn_apply_body,
        grid=(N // block_m,),
        in_specs=[
            pl.BlockSpec((block_m, F), lambda i: (i, 0)),
            pl.BlockSpec((1, F), lambda i: (0, 0)),
            pl.BlockSpec((1, F), lambda i: (0, 0)),
            pl.BlockSpec((1, F), lambda i: (0, 0)),
            pl.BlockSpec((1, F), lambda i: (0, 0)),
        ],
        out_specs=pl.BlockSpec((2, block_m, 128), lambda i: (0, i, 0)),
        out_shape=jax.ShapeDtypeStruct((2, N, 128), jnp.float32),
    )(xz, sums, sq, gamma.reshape(1, F), beta.reshape(1, F))


def _pool_out_body(hg_ref, batch_ref, w_ref, b_ref, o_ref, acc, cnt,
                   *, block_m, nsteps):
    i = pl.program_id(0)

    @pl.when(i == 0)
    def _():
        acc[...] = jnp.zeros_like(acc)
        cnt[...] = jnp.zeros_like(cnt)

    ids = batch_ref[0, 0, :]
    gids = jax.lax.broadcasted_iota(jnp.int32, (G, block_m), 0)
    onehot = (ids[None, :] == gids).astype(jnp.float32)
    acc[...] += jnp.dot(onehot, hg_ref[...],
                        preferred_element_type=jnp.float32)
    cnt[...] += jnp.sum(onehot, axis=1, keepdims=True)

    @pl.when(i == nsteps - 1)
    def _():
        pooled = acc[...] / jnp.maximum(cnt[...], 1.0)
        o_ref[...] = jnp.dot(pooled, w_ref[...],
                             preferred_element_type=jnp.float32) + b_ref[...]


def pool_out(hg, batch, w, b, block_m=2000):
    nsteps = N // block_m
    return pl.pallas_call(
        functools.partial(_pool_out_body, block_m=block_m, nsteps=nsteps),
        grid=(nsteps,),
        in_specs=[
            pl.BlockSpec((block_m, F), lambda i: (i, 0)),
            pl.BlockSpec((1, 1, block_m), lambda i: (i, 0, 0)),
            pl.BlockSpec((F, 128), lambda i: (0, 0)),
            pl.BlockSpec((1, 128), lambda i: (0, 0)),
        ],
        out_specs=pl.BlockSpec((G, 128), lambda i: (0, 0)),
        out_shape=jax.ShapeDtypeStruct((G, 128), jnp.float32),
        scratch_shapes=[
            pltpu.VMEM((G, F), jnp.float32),
            pltpu.VMEM((G, 1), jnp.float32),
        ],
    )(hg, batch.astype(jnp.int32).reshape(nsteps, 1, block_m), w,
      b.reshape(1, 128))


NS_PAD = 102400
ZS_CH = NS_PAD // 32
ZS_NCHUNK = ZS_CH // 128
ZACC_R = 10240
ZPAD_IDX = 10016

_MESH_CACHE = []


def _sc_mesh():
    if not _MESH_CACHE:
        _MESH_CACHE.append(plsc.VectorSubcoreMesh(core_axis_name="c",
                                                  subcore_axis_name="s"))
    return _MESH_CACHE[0]


def _zs_body(h2_hbm, sb_hbm, out_hbm, idx_v, rows_v, rows_v2, zb_v, acc_sh,
             sem, sem_l):
    c = lax.axis_index("c")
    s = lax.axis_index("s")
    wid = c * 16 + s
    base = wid * ZS_CH

    zb_v[...] = jnp.zeros_like(zb_v)
    @pl.loop(0, ZACC_R // 16 // 64)
    def _(j):
        pltpu.sync_copy(zb_v, acc_sh.at[pl.ds(s * (ZACC_R // 16) + j * 64, 64)])
    pltpu.async_copy(sb_hbm.at[wid], idx_v, sem).wait()
    plsc.subcore_barrier()

    rows = (rows_v, rows_v2)
    pltpu.async_copy(h2_hbm.at[pl.ds(base, 128)], rows_v, sem_l)
    for j in range(ZS_NCHUNK):
        b = j % 2
        pltpu.make_async_copy(h2_hbm.at[pl.ds(base + j * 128, 128)],
                              rows[b], sem_l).wait()
        if j < ZS_NCHUNK - 1:
            if j >= 1:
                pltpu.make_async_copy(rows[1 - b],
                                      acc_sh.at[idx_v.at[j - 1]], sem).wait()
            pltpu.async_copy(h2_hbm.at[pl.ds(base + (j + 1) * 128, 128)],
                             rows[1 - b], sem_l)
        pltpu.async_copy(rows[b], acc_sh.at[idx_v.at[j]], sem, add=True)
    pltpu.make_async_copy(rows[1], acc_sh.at[idx_v.at[ZS_NCHUNK - 2]],
                          sem).wait()
    pltpu.make_async_copy(rows[0], acc_sh.at[idx_v.at[ZS_NCHUNK - 1]],
                          sem).wait()

    plsc.subcore_barrier()
    pltpu.sync_copy(acc_sh.at[pl.ds(s * 640, 640)],
                    out_hbm.at[c, pl.ds(s * 640, 640)])


def sc_zs(h2, sub_batch_pad):
    kern = pl.kernel(
        _zs_body,
        out_type=jax.ShapeDtypeStruct((2, ZACC_R, LAT), jnp.float32),
        mesh=_sc_mesh(),
        scratch_types=[
            pltpu.VMEM((ZS_NCHUNK, 128), jnp.int32),
            pltpu.VMEM((128, LAT), jnp.float32),
            pltpu.VMEM((128, LAT), jnp.float32),
            pltpu.VMEM((64, LAT), jnp.float32),
            pltpu.VMEM_SHARED((ZACC_R, LAT), jnp.float32),
            pltpu.SemaphoreType.DMA,
            pltpu.SemaphoreType.DMA,
        ],
    )
    return kern(h2, sub_batch_pad.reshape(32, ZS_NCHUNK, 128))


def _cnt_body(sb_hbm, out_hbm, idx_v, ones_v, zb_v, acc_sh, sem):
    c = lax.axis_index("c")
    s = lax.axis_index("s")
    wid = c * 16 + s

    zb_v[...] = jnp.zeros_like(zb_v)
    @pl.loop(0, ZACC_R // 16 // 64)
    def _(j):
        pltpu.sync_copy(zb_v, acc_sh.at[pl.ds(s * (ZACC_R // 16) + j * 64, 64)])
    ones_v[...] = jnp.ones_like(ones_v)
    pltpu.async_copy(sb_hbm.at[wid], idx_v, sem).wait()
    plsc.subcore_barrier()

    for j in range(ZS_NCHUNK):
        pltpu.async_copy(ones_v, acc_sh.at[idx_v.at[j]], sem, add=True)
    for j in range(ZS_NCHUNK):
        pltpu.make_async_copy(ones_v, acc_sh.at[idx_v.at[j]], sem).wait()

    plsc.subcore_barrier()
    pltpu.sync_copy(acc_sh.at[pl.ds(s * 640, 640)],
                    out_hbm.at[c, pl.ds(s * 640, 640)])


def sc_cnt(sub_batch_pad):
    kern = pl.kernel(
        _cnt_body,
        out_type=jax.ShapeDtypeStruct((2, ZACC_R, LAT), jnp.float32),
        mesh=_sc_mesh(),
        scratch_types=[
            pltpu.VMEM((ZS_NCHUNK, 128), jnp.int32),
            pltpu.VMEM((128, LAT), jnp.float32),
            pltpu.VMEM((64, LAT), jnp.float32),
            pltpu.VMEM_SHARED((ZACC_R, LAT), jnp.float32),
            pltpu.SemaphoreType.DMA,
        ],
    )
    return kern(sub_batch_pad.reshape(32, ZS_NCHUNK, 128))


ES_PAD = 425984
AG_CH = ES_PAD // 16
CH_E = 2048
AG_NCHUNK = AG_CH // CH_E
AG_R = 10000
AG_PASSES = 5
AG_C = 128
AG_THRESH = AG_C - 16


def _agg_body(h_hbm, s_hbm, d_hbm, agg_hbm, s_v0, d_v0, s_v1, d_v1,
              sbufc, dbufc, sbuf0, dbuf0, sbuf1, dbuf1, rows0, rows1,
              zb_v, acc_sh, sem_e0, sem_e1, sem_g):
    c = lax.axis_index("c")
    s = lax.axis_index("s")
    wid = c * 16 + s
    lane = lax.iota(jnp.int32, 16)
    dummy_src = wid * 16 + lane
    dummy_dst = 10000 + s * 14 + lane
    zeros16 = jnp.zeros((16,), jnp.float32)
    ebase = s * AG_CH

    def refill_c():
        @pl.loop(0, 8)
        def _(q):
            sbufc[0, pl.ds(q * 16, 16)] = dummy_src
            dbufc[0, pl.ds(q * 16, 16)] = dummy_dst

    def stage_to(sb, db):
        @pl.loop(0, 8)
        def _(q):
            sb[0, pl.ds(q * 16, 16)] = sbufc[0, pl.ds(q * 16, 16)]
            db[0, pl.ds(q * 16, 16)] = dbufc[0, pl.ds(q * 16, 16)]
        refill_c()

    def wait_scatter(sb, db, rows):
        pltpu.make_async_copy(h_hbm.at[sb.at[0]], rows, sem_g).wait()
        pltpu.sync_copy(rows, acc_sh.at[db.at[0]], add=True)

    @pl.loop(0, 32)
    def _(r):
        @pl.loop(0, 8)
        def _(q):
            zb_v[r, pl.ds(q * 16, 16)] = zeros16
    refill_c()

    for p in range(AG_PASSES):
        lo = c * (AG_PASSES * AG_R) + p * AG_R

        @pl.loop(0, ZACC_R // 16 // 32)
        def _(j):
            pltpu.sync_copy(zb_v,
                            acc_sh.at[pl.ds(s * (ZACC_R // 16) + j * 32, 32)])
        plsc.subcore_barrier()

        pltpu.async_copy(s_hbm.at[pl.ds(ebase, CH_E)], s_v0, sem_e0)
        pltpu.async_copy(d_hbm.at[pl.ds(ebase, CH_E)], d_v0, sem_e0)

        def scan_chunk(ci, carry, sv_ref, dv_ref, nsv_ref, ndv_ref,
                       sem_cur, sem_nxt):
            pltpu.make_async_copy(s_hbm.at[pl.ds(ebase + ci * CH_E, CH_E)],
                                  sv_ref, sem_cur).wait()
            pltpu.make_async_copy(d_hbm.at[pl.ds(ebase + ci * CH_E, CH_E)],
                                  dv_ref, sem_cur).wait()

            @pl.when(ci < AG_NCHUNK - 1)
            def _():
                pltpu.async_copy(
                    s_hbm.at[pl.ds(ebase + (ci + 1) * CH_E, CH_E)],
                    nsv_ref, sem_nxt)
                pltpu.async_copy(
                    d_hbm.at[pl.ds(ebase + (ci + 1) * CH_E, CH_E)],
                    ndv_ref, sem_nxt)

            def body(i, cf):
                cnt, f = cf
                dv = dv_ref[pl.ds(i * 16, 16)]
                sv = sv_ref[pl.ds(i * 16, 16)]
                ldv = dv - lo
                m = (ldv >= 0) & (ldv < AG_R)
                pos = cnt + plsc.cumsum(m.astype(jnp.int32)) - 1
                plsc.store_scatter(sbufc, [pos - pos, pos], sv, mask=m)
                plsc.store_scatter(dbufc, [pos - pos, pos], ldv, mask=m)
                newcnt = cnt + jnp.max(plsc.all_reduce_population_count(m))
                do_flush = newcnt >= AG_THRESH
                cp = lax.bitwise_and(f, 1)

                @pl.when(do_flush & (cp == 0))
                def _():
                    @pl.when(f > 0)
                    def _():
                        wait_scatter(sbuf1, dbuf1, rows1)
                    stage_to(sbuf0, dbuf0)
                    pltpu.async_copy(h_hbm.at[sbuf0.at[0]], rows0, sem_g)

                @pl.when(do_flush & (cp == 1))
                def _():
                    wait_scatter(sbuf0, dbuf0, rows0)
                    stage_to(sbuf1, dbuf1)
                    pltpu.async_copy(h_hbm.at[sbuf1.at[0]], rows1, sem_g)

                return (jnp.where(do_flush, 0, newcnt),
                        jnp.where(do_flush, f + 1, f))

            return lax.fori_loop(0, CH_E // 16, body, carry)

        def chunk_body(ci, carry):
            return lax.cond(
                lax.bitwise_and(ci, 1) == 0,
                lambda cr: scan_chunk(ci, cr, s_v0, d_v0, s_v1, d_v1,
                                      sem_e0, sem_e1),
                lambda cr: scan_chunk(ci, cr, s_v1, d_v1, s_v0, d_v0,
                                      sem_e1, sem_e0),
                carry)

        cnt, f = lax.fori_loop(0, AG_NCHUNK, chunk_body,
                               (jnp.int32(0), jnp.int32(0)))
        cp = lax.bitwise_and(f, 1)

        @pl.when((f > 0) & (cp == 1))
        def _():
            wait_scatter(sbuf0, dbuf0, rows0)

        @pl.when((f > 0) & (cp == 0))
        def _():
            wait_scatter(sbuf1, dbuf1, rows1)

        stage_to(sbuf0, dbuf0)
        pltpu.async_copy(h_hbm.at[sbuf0.at[0]], rows0, sem_g)
        wait_scatter(sbuf0, dbuf0, rows0)
        plsc.subcore_barrier()

        @pl.when(s < 15)
        def _():
            pltpu.sync_copy(acc_sh.at[pl.ds(s * 640, 640)],
                            agg_hbm.at[pl.ds(lo + s * 640, 640)])

        @pl.when(s == 15)
        def _():
            pltpu.sync_copy(acc_sh.at[pl.ds(9600, 400)],
                            agg_hbm.at[pl.ds(lo + 9600, 400)])

        plsc.subcore_barrier()


def sc_agg(h, s_pad, d_pad):
    kern = pl.kernel(
        _agg_body,
        compiler_params=pltpu.CompilerParams(needs_layout_passes=False),
        out_type=jax.ShapeDtypeStruct((NS, LAT), jnp.float32),
        mesh=_sc_mesh(),
        scratch_types=[
            pltpu.VMEM((CH_E,), jnp.int32),
            pltpu.VMEM((CH_E,), jnp.int32),
            pltpu.VMEM((CH_E,), jnp.int32),
            pltpu.VMEM((CH_E,), jnp.int32),
            pltpu.VMEM((1, 128), jnp.int32),
            pltpu.VMEM((1, 128), jnp.int32),
            pltpu.VMEM((1, 128), jnp.int32),
            pltpu.VMEM((1, 128), jnp.int32),
            pltpu.VMEM((1, 128), jnp.int32),
            pltpu.VMEM((1, 128), jnp.int32),
            pltpu.VMEM((AG_C, LAT), jnp.float32),
            pltpu.VMEM((AG_C, LAT), jnp.float32),
            pltpu.VMEM((32, LAT), jnp.float32),
            pltpu.VMEM_SHARED((ZACC_R, LAT), jnp.float32),
            pltpu.SemaphoreType.DMA,
            pltpu.SemaphoreType.DMA,
            pltpu.SemaphoreType.DMA,
        ],
    )
    return kern(h, s_pad, d_pad)


E_PAD = 163840
W_CH = E_PAD // 32
M_CH = E_PAD // 16
M_SUB = 1024


def _w_body(px_h, py_h, pz_h, es_h, ed_h, w_h, px_v, py_v, pz_v,
            es_v, ed_v, w_v, sem):
    c = lax.axis_index("c")
    s = lax.axis_index("s")
    wid = c * 16 + s
    base = wid * W_CH
    pltpu.async_copy(px_h, px_v, sem).wait()
    pltpu.async_copy(py_h, py_v, sem).wait()
    pltpu.async_copy(pz_h, pz_v, sem).wait()
    pltpu.async_copy(es_h.at[pl.ds(base, W_CH)], es_v, sem).wait()
    pltpu.async_copy(ed_h.at[pl.ds(base, W_CH)], ed_v, sem).wait()

    @pl.loop(0, W_CH // 16)
    def _(i):
        esv = es_v[pl.ds(i * 16, 16)]
        edv = ed_v[pl.ds(i * 16, 16)]
        dx = plsc.load_gather(px_v, [esv]) - plsc.load_gather(px_v, [edv])
        dy = plsc.load_gather(py_v, [esv]) - plsc.load_gather(py_v, [edv])
        dz = plsc.load_gather(pz_v, [esv]) - plsc.load_gather(pz_v, [edv])
        w_v[pl.ds(i * 16, 16)] = jnp.exp(-(dx * dx + dy * dy + dz * dz))

    pltpu.sync_copy(w_v, w_h.at[pl.ds(base, W_CH)])


def sc_w(px, py, pz, es_w, ed_w):
    kern = pl.kernel(
        _w_body,
        compiler_params=pltpu.CompilerParams(needs_layout_passes=False),
        out_type=jax.ShapeDtypeStruct((E_PAD,), jnp.float32),
        mesh=_sc_mesh(),
        scratch_types=[
            pltpu.VMEM((N,), jnp.float32),
            pltpu.VMEM((N,), jnp.float32),
            pltpu.VMEM((N,), jnp.float32),
            pltpu.VMEM((W_CH,), jnp.int32),
            pltpu.VMEM((W_CH,), jnp.int32),
            pltpu.VMEM((W_CH,), jnp.float32),
            pltpu.SemaphoreType.DMA,
        ],
    )
    return kern(px, py, pz, es_w, ed_w)


def _m_body(xn_h, es_h, ed_h, w_h, m_h, es_c, ed_c, w_c, rows0, rows1,
            zb_v, acc_sh, sem_e, sem_g, sem_s):
    c = lax.axis_index("c")
    s = lax.axis_index("s")
    zeros16 = jnp.zeros((16,), jnp.float32)
    rows = (rows0, rows1)

    @pl.loop(0, 32)
    def _(r):
        @pl.loop(0, 8)
        def _(q):
            zb_v[r, pl.ds(q * 16, 16)] = zeros16

    @pl.loop(0, ZACC_R // 16 // 32)
    def _(j):
        pltpu.sync_copy(zb_v, acc_sh.at[pl.ds(s * (ZACC_R // 16) + j * 32, 32)])
    plsc.subcore_barrier()

    def scale(rbuf, base):
        @pl.loop(0, 128)
        def _(r):
            wsp = plsc.load_gather(w_c, [jnp.full((16,), base + r, jnp.int32)])

            @pl.loop(0, 8)
            def _(k):
                rbuf[r, pl.ds(k * 16, 16)] = rbuf[r, pl.ds(k * 16, 16)] * wsp

    @pl.loop(0, M_CH // M_SUB)
    def _(q):
        row0 = s * (M_CH // 128) + q * (M_SUB // 128)
        pltpu.async_copy(es_h.at[pl.ds(row0, M_SUB // 128)], es_c, sem_e)
        pltpu.async_copy(ed_h.at[pl.ds(row0, M_SUB // 128)], ed_c, sem_e)
        pltpu.async_copy(w_h.at[pl.ds(s * M_CH + q * M_SUB, M_SUB)],
                         w_c, sem_e)
        pltpu.make_async_copy(es_h.at[pl.ds(row0, M_SUB // 128)],
                              es_c, sem_e).wait()
        pltpu.make_async_copy(ed_h.at[pl.ds(row0, M_SUB // 128)],
                              ed_c, sem_e).wait()
        pltpu.make_async_copy(w_h.at[pl.ds(s * M_CH + q * M_SUB, M_SUB)],
                              w_c, sem_e).wait()

        pltpu.async_copy(xn_h.at[c].at[es_c.at[0]], rows0, sem_g)
        for j in range(M_SUB // 128):
            b = j % 2
            pltpu.make_async_copy(xn_h.at[c].at[es_c.at[j]],
                                  rows[b], sem_g).wait()
            if j < M_SUB // 128 - 1:
                if j >= 1:
                    pltpu.make_async_copy(rows[1 - b],
                                          acc_sh.at[ed_c.at[j - 1]],
                                          sem_s).wait()
                pltpu.async_copy(xn_h.at[c].at[es_c.at[j + 1]],
                                 rows[1 - b], sem_g)
            scale(rows[b], j * 128)
            pltpu.async_copy(rows[b], acc_sh.at[ed_c.at[j]], sem_s, add=True)
        pltpu.make_async_copy(rows[0], acc_sh.at[ed_c.at[6]], sem_s).wait()
        pltpu.make_async_copy(rows[1], acc_sh.at[ed_c.at[7]], sem_s).wait()

    plsc.subcore_barrier()

    @pl.when(s < 15)
    def _():
        pltpu.sync_copy(acc_sh.at[pl.ds(s * 640, 640)],
                        m_h.at[c, pl.ds(s * 640, 640)])

    @pl.when(s == 15)
    def _():
        pltpu.sync_copy(acc_sh.at[pl.ds(9600, 640)],
                        m_h.at[c, pl.ds(9600, 640)])

    plsc.subcore_barrier()


def sc_m(xn2, es2d, ed2d, w):
    kern = pl.kernel(
        _m_body,
        compiler_params=pltpu.CompilerParams(needs_layout_passes=False),
        out_type=jax.ShapeDtypeStruct((2, ZACC_R, 128), jnp.float32),
        mesh=_sc_mesh(),
        scratch_types=[
            pltpu.VMEM((M_SUB // 128, 128), jnp.int32),
            pltpu.VMEM((M_SUB // 128, 128), jnp.int32),
            pltpu.VMEM((M_SUB,), jnp.float32),
            pltpu.VMEM((128, 128), jnp.float32),
            pltpu.VMEM((128, 128), jnp.float32),
            pltpu.VMEM((32, 128), jnp.float32),
            pltpu.VMEM_SHARED((ZACC_R, 128), jnp.float32),
            pltpu.SemaphoreType.DMA,
            pltpu.SemaphoreType.DMA,
            pltpu.SemaphoreType.DMA,
        ],
    )
    return kern(xn2, es2d, ed2d, w)


def kernel(x, adj_t, pos, batch, sub_x, sub_adj_t, sub_batch, edge_index,
           batch_lengths, edge_batch, We1, be1, We2, be2, gamma, beta,
           Wg1, bg1, Wg2, bg2):
    h = mm_bias(sub_x, We1, be1, relu=True)
    s_pad = jnp.pad(sub_adj_t[0].astype(jnp.int32), (0, ES_PAD - ES))
    d_pad = jnp.pad(sub_adj_t[1].astype(jnp.int32), (0, ES_PAD - ES),
                    constant_values=2 ** 30)
    agg = sc_agg(h, s_pad, d_pad)
    h2 = mm_bias(agg, We2, be2, relu=True, out_rows=NS_PAD)
    sb_pad = jnp.pad(sub_batch.astype(jnp.int32), (0, NS_PAD - NS),
                     constant_values=ZPAD_IDX)
    zcnt = sc_zs(h2, sb_pad)
    cnts = sc_cnt(sb_pad)
    xz, sums, sq = bn_stats(x, zcnt, cnts)
    xn2 = bn_apply(xz, sums, sq, gamma, beta)
    es = edge_index[0].astype(jnp.int32)
    ed = edge_index[1].astype(jnp.int32)
    es_w = jnp.pad(es, (0, E_PAD - E))
    ed_w = jnp.pad(ed, (0, E_PAD - E))
    w = sc_w(pos[:, 0], pos[:, 1], pos[:, 2], es_w, ed_w)
    ed_m = jnp.pad(ed, (0, E_PAD - E), constant_values=ZPAD_IDX)
    m2 = sc_m(xn2, es_w.reshape(E_PAD // 128, 128),
              ed_m.reshape(E_PAD // 128, 128), w)
    hg = mm_split(m2, Wg1, bg1)
    return pool_out(hg, batch, Wg2, bg2)
